# baseline (speedup 1.0000x reference)
import functools
import math

import jax
import jax.numpy as jnp
from jax import lax
from jax.experimental import pallas as pl
from jax.experimental.pallas import tpu as pltpu
from jax.experimental.pallas import tpu_sc as plsc

B, T, C, H, E, K, FF = 1, 2048, 768, 12, 16, 2, 3072
HD = C // H
BT = 256
BR = 128
NB = T * K // BR + E
NR = NB * BR
NEG = -1e30
NT1 = 16
TPT = T // NT1
NW = 32
TPW = T // NW
HTPW = TPW // 2


BKV = 1024
H2 = H // 2
NA = T // BT
NBS = H2 * (T // BT) * (T // BKV)
NC = NA + NBS


def _fused_body(x_ref, ln1w_ref, ln1b_ref, wq_ref, bq_ref, wk_ref, bk_ref,
                wv_ref, bv_ref, eps_ref, wo_ref, bo_ref, ln2w_ref, ln2b_ref,
                wg_ref, bg_ref, wn_ref, bn_ref,
                hs_ref, noisy_ref, gate_ref, noisyT_ref,
                qs, ks, vs, ys, acc_ref, m_ref, l_ref):
    s = pl.program_id(0)

    @pl.when(s < NA)
    def _():
        i = s
        x = x_ref[...]
        mu = jnp.mean(x, -1, keepdims=True)
        xc = x - mu
        var = jnp.mean(xc * xc, -1, keepdims=True)
        xn = xc * jax.lax.rsqrt(var + 1e-5) * ln1w_ref[...] + ln1b_ref[...]
        q = jnp.dot(xn, wq_ref[...], preferred_element_type=jnp.float32) + bq_ref[...]
        k = jnp.dot(xn, wk_ref[...], preferred_element_type=jnp.float32) + bk_ref[...]
        v = jnp.dot(xn, wv_ref[...], preferred_element_type=jnp.float32) + bv_ref[...]
        rsl = pl.ds(i * BT, BT)
        for h2 in range(H2):
            csl = slice(h2 * 2 * HD, (h2 + 1) * 2 * HD)
            qs[h2, rsl, :] = q[:, csl]
            ks[h2, rsl, :] = k[:, csl]
            vs[h2, rsl, :] = v[:, csl]

    @pl.when((s >= NA) & (s < NC))
    def _():
        idx = s - NA
        per_h = (T // BT) * (T // BKV)
        h2d = idx // per_h
        r = idx % per_h
        i = r // (T // BKV)
        kb = r % (T // BKV)
        last = i // (BKV // BT)

        @pl.when(kb <= last)
        def _():
            q2 = qs[h2d, pl.ds(i * BT, BT), :]
            k2 = ks[h2d, pl.ds(kb * BKV, BKV), :]
            v2 = vs[h2d, pl.ds(kb * BKV, BKV), :]
            rows = jax.lax.broadcasted_iota(jnp.int32, (BT, BKV), 0) + i * BT
            cols = jax.lax.broadcasted_iota(jnp.int32, (BT, BKV), 1) + kb * BKV
            causal = cols <= rows
            first = kb == 0
            for hh in range(2):
                sl = slice(hh * HD, (hh + 1) * HD)
                sc = jax.lax.dot_general(q2[:, sl], k2[:, sl],
                                         (((1,), (1,)), ((), ())),
                                         preferred_element_type=jnp.float32)
                sc = sc * (1.0 / math.sqrt(HD))
                sc = jnp.where(causal, sc, NEG)
                m_old = jnp.where(first, -3e38, m_ref[:, hh * HD:hh * HD + 1])
                l_old = jnp.where(first, 0.0, l_ref[:, hh * HD:hh * HD + 1])
                acc_old = jnp.where(first, 0.0, acc_ref[:, sl])
                m_cur = jnp.max(sc, -1, keepdims=True)
                m_new = jnp.maximum(m_old, m_cur)
                scale = jnp.exp(m_old - m_new)
                p = jnp.exp(sc - m_new)
                l_new = l_old * scale + jnp.sum(p, -1, keepdims=True)
                acc_new = acc_old * scale + jnp.dot(
                    p, v2[:, sl], preferred_element_type=jnp.float32)
                m_ref[:, sl] = jnp.broadcast_to(m_new, (BT, HD))
                l_ref[:, sl] = jnp.broadcast_to(l_new, (BT, HD))
                acc_ref[:, sl] = acc_new

                @pl.when(kb == last)
                def _():
                    ys[h2d, pl.ds(i * BT, BT), sl] = acc_new / l_new

    @pl.when(s >= NC)
    def _():
        i = s - NC
        y = jnp.concatenate(
            [ys[h2, pl.ds(i * BT, BT), :] for h2 in range(H2)], axis=1)
        h = x_ref[...] + jnp.dot(y, wo_ref[...],
                                 preferred_element_type=jnp.float32) + bo_ref[...]
        mu = jnp.mean(h, -1, keepdims=True)
        hc = h - mu
        var = jnp.mean(hc * hc, -1, keepdims=True)
        hs = hc * jax.lax.rsqrt(var + 1e-5) * ln2w_ref[...] + ln2b_ref[...]
        hs_ref[...] = hs
        g = jnp.dot(hs, wg_ref[...], preferred_element_type=jnp.float32) + bg_ref[...]
        nz = jnp.dot(hs, wn_ref[...], preferred_element_type=jnp.float32) + bn_ref[...]
        sp = jnp.maximum(nz, 0.0) + jnp.log1p(jnp.exp(-jnp.abs(nz)))
        gate_ref[...] = g
        noisy = g + eps_ref[...] * sp
        noisy_ref[...] = noisy
        noisyT_ref[...] = noisy.T


def _ffn_body(bex_ref, hs_ref, we1_ref, be1_ref, we2_ref, be2_ref, out_ref):
    t = jnp.dot(hs_ref[...], we1_ref[0],
                preferred_element_type=jnp.float32) + be1_ref[0]
    t = 0.5 * t * (1.0 + jax.lax.erf(t * (1.0 / math.sqrt(2.0))))
    out_ref[...] = jnp.dot(t, we2_ref[0],
                           preferred_element_type=jnp.float32) + be2_ref[0]


def _iota16():
    return jax.lax.broadcasted_iota(jnp.int32, (16,), 0)


def _cumsum16(x, tmp_v, lanes):
    zf = jnp.zeros((16,), jnp.float32)
    for k in (1, 2, 4, 8):
        tmp_v[...] = x
        sh = plsc.load_gather(tmp_v, [jnp.maximum(lanes - k, 0)])
        x = x + jnp.where(lanes >= k, sh, zf)
    return x


def _router_body(nlT_ref, hs_ref,
                 pos0_ref, pos1_ref, w0_ref, w1_ref, bex_ref, sorted_ref,
                 nl_v, hs_v, pose_v, wtsl_v, e_v, rank_v, cnt_v, cnts_all_v,
                 offbase_v, csum_v, bex_v, tmp_v, shared, sem_hs, sem_sc):
    sid = lax.axis_index("s")
    base_tok = sid * TPT
    hs_cp = pltpu.make_async_copy(hs_ref.at[pl.ds(base_tok, TPT)], hs_v, sem_hs)
    hs_cp.start()
    pltpu.sync_copy(nlT_ref.at[:, pl.ds(base_tok, TPT)], nl_v)

    lanes = _iota16()
    cnt = [jnp.float32(0)] * E
    for g in range(TPT // 16):
        sl = pl.ds(g * 16, 16)
        vs = [nl_v[e, sl] for e in range(E)]
        m1 = functools.reduce(jnp.maximum, vs)
        e1v = jnp.full((16,), E, jnp.int32)
        for e in range(E):
            e1v = jnp.minimum(e1v, jnp.where(vs[e] == m1, e, E))
        vs2 = [jnp.where(e1v == e, NEG, vs[e]) for e in range(E)]
        m2 = functools.reduce(jnp.maximum, vs2)
        e2v = jnp.full((16,), E, jnp.int32)
        for e in range(E):
            e2v = jnp.minimum(e2v, jnp.where(vs2[e] == m2, e, E))
        a = jnp.exp(m2 - m1)
        wtsl_v[0, sl] = 1.0 / (1.0 + a)
        wtsl_v[1, sl] = a / (1.0 + a)
        rank0 = jnp.zeros((16,), jnp.float32)
        rank1 = jnp.zeros((16,), jnp.float32)
        for e in range(E):
            m0 = e1v == e
            m1b = e2v == e
            mf = jnp.where(jnp.logical_or(m0, m1b), 1.0, 0.0)
            incl = _cumsum16(mf, tmp_v, lanes)
            r = cnt[e] + (incl - mf)
            rank0 = jnp.where(m0, r, rank0)
            rank1 = jnp.where(m1b, r, rank1)
            cnt[e] = cnt[e] + incl[15]
        e_v[0, sl] = e1v
        e_v[1, sl] = e2v
        rank_v[0, sl] = rank0.astype(jnp.int32)
        rank_v[1, sl] = rank1.astype(jnp.int32)

    cv = jnp.zeros((16,), jnp.float32)
    for e in range(E):
        cv = jnp.where(lanes == e, cnt[e], cv)
    cnt_v[pl.ds(0, 16)] = cv
    zf16 = jnp.zeros((16,), jnp.float32)
    for j in range(1, 8):
        cnt_v[pl.ds(j * 16, 16)] = zf16
    pltpu.sync_copy(cnt_v, shared.at[sid])
    plsc.subcore_barrier()
    pltpu.sync_copy(shared, cnts_all_v)

    sidv = jnp.full((16,), sid, jnp.int32)
    basev = jnp.zeros((16,), jnp.float32)
    totv = jnp.zeros((16,), jnp.float32)
    zf = jnp.zeros((16,), jnp.float32)
    for w2 in range(NT1):
        row = cnts_all_v[w2, pl.ds(0, 16)]
        basev = basev + jnp.where(jnp.full((16,), w2, jnp.int32) < sidv, row, zf)
        totv = totv + row
    pc = ((totv.astype(jnp.int32) + (BR - 1)) >> 7) << 7
    pcf = pc.astype(jnp.float32)
    csum = _cumsum16(pcf, tmp_v, lanes)
    off = csum - pcf
    offbase_v[...] = (off + basev).astype(jnp.int32)
    csum_v[...] = csum.astype(jnp.int32)

    for g in range(TPT // 16):
        sl = pl.ds(g * 16, 16)
        pose_v[0, sl] = plsc.load_gather(offbase_v, [e_v[0, sl]]) + rank_v[0, sl]
        pose_v[1, sl] = plsc.load_gather(offbase_v, [e_v[1, sl]]) + rank_v[1, sl]
    pltpu.sync_copy(pose_v.at[0], pos0_ref.at[pl.ds(base_tok, TPT)])
    pltpu.sync_copy(pose_v.at[1], pos1_ref.at[pl.ds(base_tok, TPT)])
    pltpu.sync_copy(wtsl_v.at[0], w0_ref.at[pl.ds(base_tok, TPT)])
    pltpu.sync_copy(wtsl_v.at[1], w1_ref.at[pl.ds(base_tok, TPT)])

    @pl.when(sid == 0)
    def _():
        csumv = csum_v[...]
        for j in range(NB // 16):
            bvec = lanes + j * 16
            acc = jnp.zeros((16,), jnp.int32)
            for e in range(E):
                end_e = csumv[e] >> 7
                acc = acc + jnp.where(bvec >= end_e, 1, 0).astype(jnp.int32)
            bex_v[pl.ds(j * 16, 16)] = jnp.minimum(acc, E - 1)
        pltpu.sync_copy(bex_v, bex_ref)

    hs_cp.wait()
    pltpu.async_copy(hs_v, sorted_ref.at[pose_v.at[0]], sem_sc).wait()
    pltpu.async_copy(hs_v, sorted_ref.at[pose_v.at[1]], sem_sc).wait()


def _combine_body(x_ref, outc_ref, pos0_ref, pos1_ref, w0_ref, w1_ref,
                  final_ref, pos0_v, pos1_v, w0_v, w1_v, a_v, b_v, x_v, f_v,
                  sem):
    wid = lax.axis_index("s") * 2 + lax.axis_index("c")
    base_tok = wid * TPW
    pltpu.sync_copy(pos0_ref.at[pl.ds(base_tok, TPW)], pos0_v)
    pltpu.sync_copy(pos1_ref.at[pl.ds(base_tok, TPW)], pos1_v)
    pltpu.sync_copy(w0_ref.at[pl.ds(base_tok, TPW)], w0_v)
    pltpu.sync_copy(w1_ref.at[pl.ds(base_tok, TPW)], w1_v)
    for jh in range(2):
        tok0 = base_tok + jh * HTPW
        pltpu.async_copy(outc_ref.at[pos0_v.at[pl.ds(jh * HTPW, HTPW)]],
                         a_v, sem).wait()
        pltpu.async_copy(outc_ref.at[pos1_v.at[pl.ds(jh * HTPW, HTPW)]],
                         b_v, sem).wait()
        pltpu.sync_copy(x_ref.at[pl.ds(tok0, HTPW)], x_v)

        def body(t, _):
            ti = jnp.full((16,), jh * HTPW + t, jnp.int32)
            w0 = plsc.load_gather(w0_v, [ti])
            w1 = plsc.load_gather(w1_v, [ti])
            for c in range(C // 16):
                sl = pl.ds(c * 16, 16)
                f_v[t, sl] = x_v[t, sl] + w0 * a_v[t, sl] + w1 * b_v[t, sl]
            return 0

        lax.fori_loop(0, HTPW, body, 0)
        pltpu.sync_copy(f_v, final_ref.at[pl.ds(tok0, HTPW)])


def kernel(hidden_states, ln1_w, ln1_b, ln2_w, ln2_b, Wq, bq, Wk, bk, Wv, bv,
           Wo, bo, Wg, bg, Wn, bn_, We1, be1, We2, be2, noise_eps):
    x2d = hidden_states.reshape(T, C)
    full = lambda r, c: pl.BlockSpec((r, c), lambda *_: (0, 0))

    def _xrow(s):
        return (jnp.where(s < NA, s,
                          jnp.where(s >= NC, s - NC, NA - 1)), 0)

    def _crow(s):
        return (jnp.where(s >= NC, s - NC, 0), 0)

    def _ccol(s):
        return (0, jnp.where(s >= NC, s - NC, 0))

    hs, noisy, gate, noisy_t = pl.pallas_call(
        _fused_body,
        grid=(NC + NA,),
        in_specs=[
            pl.BlockSpec((BT, C), _xrow),
            full(1, C), full(1, C),
            full(C, C), full(1, C), full(C, C), full(1, C),
            full(C, C), full(1, C),
            pl.BlockSpec((BT, E), _crow),
            full(C, C), full(1, C), full(1, C), full(1, C),
            full(C, E), full(1, E), full(C, E), full(1, E),
        ],
        out_specs=[
            pl.BlockSpec((BT, C), _crow),
            pl.BlockSpec((BT, E), _crow),
            pl.BlockSpec((BT, E), _crow),
            pl.BlockSpec((E, BT), _ccol),
        ],
        out_shape=[
            jax.ShapeDtypeStruct((T, C), jnp.float32),
            jax.ShapeDtypeStruct((T, E), jnp.float32),
            jax.ShapeDtypeStruct((T, E), jnp.float32),
            jax.ShapeDtypeStruct((E, T), jnp.float32),
        ],
        scratch_shapes=[
            pltpu.VMEM((H2, T, 2 * HD), jnp.float32),
            pltpu.VMEM((H2, T, 2 * HD), jnp.float32),
            pltpu.VMEM((H2, T, 2 * HD), jnp.float32),
            pltpu.VMEM((H2, T, 2 * HD), jnp.float32),
            pltpu.VMEM((BT, 2 * HD), jnp.float32),
            pltpu.VMEM((BT, 2 * HD), jnp.float32),
            pltpu.VMEM((BT, 2 * HD), jnp.float32),
        ],
    )(x2d, ln1_w.reshape(1, C), ln1_b.reshape(1, C), Wq, bq.reshape(1, C),
      Wk, bk.reshape(1, C), Wv, bv.reshape(1, C), noise_eps,
      Wo, bo.reshape(1, C), ln2_w.reshape(1, C), ln2_b.reshape(1, C),
      Wg, bg.reshape(1, E), Wn, bn_.reshape(1, E))

    router = functools.partial(
        pl.kernel,
        out_type=[
            jax.ShapeDtypeStruct((T,), jnp.int32),
            jax.ShapeDtypeStruct((T,), jnp.int32),
            jax.ShapeDtypeStruct((T,), jnp.float32),
            jax.ShapeDtypeStruct((T,), jnp.float32),
            jax.ShapeDtypeStruct((NB,), jnp.int32),
            jax.ShapeDtypeStruct((NR, C), jnp.float32),
        ],
        mesh=plsc.VectorSubcoreMesh(core_axis_name="c", subcore_axis_name="s",
                                    num_cores=1),
        compiler_params=pltpu.CompilerParams(needs_layout_passes=False),
        scratch_types=[
            pltpu.VMEM((E, TPT), jnp.float32),
            pltpu.VMEM((TPT, C), jnp.float32),
            pltpu.VMEM((2, TPT), jnp.int32),
            pltpu.VMEM((2, TPT), jnp.float32),
            pltpu.VMEM((2, TPT), jnp.int32),
            pltpu.VMEM((2, TPT), jnp.int32),
            pltpu.VMEM((128,), jnp.float32),
            pltpu.VMEM((NT1, 128), jnp.float32),
            pltpu.VMEM((16,), jnp.int32),
            pltpu.VMEM((16,), jnp.int32),
            pltpu.VMEM((NB,), jnp.int32),
            pltpu.VMEM((16,), jnp.float32),
            pltpu.VMEM_SHARED((NT1, 128), jnp.float32),
            pltpu.SemaphoreType.DMA,
            pltpu.SemaphoreType.DMA,
        ],
    )(_router_body)
    pos0, pos1, w0, w1, bex, sorted_rows = router(noisy_t, hs)

    grid_spec = pltpu.PrefetchScalarGridSpec(
        num_scalar_prefetch=1,
        grid=(NB,),
        in_specs=[
            pl.BlockSpec((BR, C), lambda g, b: (g, 0)),
            pl.BlockSpec((1, C, FF), lambda g, b: (b[g], 0, 0)),
            pl.BlockSpec((1, 1, FF), lambda g, b: (b[g], 0, 0)),
            pl.BlockSpec((1, FF, C), lambda g, b: (b[g], 0, 0)),
            pl.BlockSpec((1, 1, C), lambda g, b: (b[g], 0, 0)),
        ],
        out_specs=pl.BlockSpec((BR, C), lambda g, b: (g, 0)),
    )
    outc = pl.pallas_call(
        _ffn_body,
        grid_spec=grid_spec,
        out_shape=jax.ShapeDtypeStruct((NR, C), jnp.float32),
    )(bex, sorted_rows, We1, be1.reshape(E, 1, FF),
      We2, be2.reshape(E, 1, C))

    combine = functools.partial(
        pl.kernel,
        out_type=jax.ShapeDtypeStruct((T, C), jnp.float32),
        mesh=plsc.VectorSubcoreMesh(core_axis_name="c", subcore_axis_name="s"),
        compiler_params=pltpu.CompilerParams(needs_layout_passes=False),
        scratch_types=[
            pltpu.VMEM((TPW,), jnp.int32),
            pltpu.VMEM((TPW,), jnp.int32),
            pltpu.VMEM((TPW,), jnp.float32),
            pltpu.VMEM((TPW,), jnp.float32),
            pltpu.VMEM((HTPW, C), jnp.float32),
            pltpu.VMEM((HTPW, C), jnp.float32),
            pltpu.VMEM((HTPW, C), jnp.float32),
            pltpu.VMEM((HTPW, C), jnp.float32),
            pltpu.SemaphoreType.DMA,
        ],
    )(_combine_body)
    final = combine(x2d, outc, pos0, pos1, w0, w1)

    return (final.reshape(B, T, C), noisy, gate)


# speedup vs baseline: 1.0363x; 1.0363x over previous
import functools
import math

import jax
import jax.numpy as jnp
from jax import lax
from jax.experimental import pallas as pl
from jax.experimental.pallas import tpu as pltpu
from jax.experimental.pallas import tpu_sc as plsc

B, T, C, H, E, K, FF = 1, 2048, 768, 12, 16, 2, 3072
HD = C // H
BT = 256
BR = 128
NB = T * K // BR + E
NR = NB * BR
NEG = -1e30
NT1 = 16
TPT = T // NT1
NW = 32
TPW = T // NW
HTPW = TPW // 2


H2 = H // 2
NA = T // BT
NBS = H2 * (T // BT)
NC = NA + NBS


def _fused_body(x_ref, ln1w_ref, ln1b_ref, wq_ref, bq_ref, wk_ref, bk_ref,
                wv_ref, bv_ref, eps_ref, wo_ref, bo_ref, ln2w_ref, ln2b_ref,
                wg_ref, bg_ref, wn_ref, bn_ref,
                hs_ref, noisy_ref, gate_ref, noisyT_ref,
                qs, ks, vs, ys):
    s = pl.program_id(0)

    @pl.when(s < NA)
    def _():
        i = s
        x = x_ref[...]
        mu = jnp.mean(x, -1, keepdims=True)
        xc = x - mu
        var = jnp.mean(xc * xc, -1, keepdims=True)
        xn = xc * jax.lax.rsqrt(var + 1e-5) * ln1w_ref[...] + ln1b_ref[...]
        q = jnp.dot(xn, wq_ref[...], preferred_element_type=jnp.float32) + bq_ref[...]
        k = jnp.dot(xn, wk_ref[...], preferred_element_type=jnp.float32) + bk_ref[...]
        v = jnp.dot(xn, wv_ref[...], preferred_element_type=jnp.float32) + bv_ref[...]
        rsl = pl.ds(i * BT, BT)
        for h2 in range(H2):
            csl = slice(h2 * 2 * HD, (h2 + 1) * 2 * HD)
            qs[h2, rsl, :] = q[:, csl]
            ks[h2, rsl, :] = k[:, csl]
            vs[h2, rsl, :] = v[:, csl]

    @pl.when((s >= NA) & (s < NC))
    def _():
        idx = s - NA
        h2d = idx // (T // BT)
        i = idx % (T // BT)
        q2 = qs[h2d, pl.ds(i * BT, BT), :]
        k2 = ks[h2d]
        v2 = vs[h2d]
        rows = jax.lax.broadcasted_iota(jnp.int32, (BT, T), 0) + i * BT
        cols = jax.lax.broadcasted_iota(jnp.int32, (BT, T), 1)
        causal = cols <= rows
        for hh in range(2):
            sl = slice(hh * HD, (hh + 1) * HD)
            sc = jax.lax.dot_general(q2[:, sl], k2[:, sl],
                                     (((1,), (1,)), ((), ())),
                                     preferred_element_type=jnp.float32)
            sc = sc * (1.0 / math.sqrt(HD))
            sc = jnp.where(causal, sc, NEG)
            m = jnp.max(sc, -1, keepdims=True)
            p = jnp.exp(sc - m)
            p = p / jnp.sum(p, -1, keepdims=True)
            ys[h2d, pl.ds(i * BT, BT), sl] = jnp.dot(
                p, v2[:, sl], preferred_element_type=jnp.float32)

    @pl.when(s >= NC)
    def _():
        i = s - NC
        y = jnp.concatenate(
            [ys[h2, pl.ds(i * BT, BT), :] for h2 in range(H2)], axis=1)
        h = x_ref[...] + jnp.dot(y, wo_ref[...],
                                 preferred_element_type=jnp.float32) + bo_ref[...]
        mu = jnp.mean(h, -1, keepdims=True)
        hc = h - mu
        var = jnp.mean(hc * hc, -1, keepdims=True)
        hs = hc * jax.lax.rsqrt(var + 1e-5) * ln2w_ref[...] + ln2b_ref[...]
        hs_ref[...] = hs
        g = jnp.dot(hs, wg_ref[...], preferred_element_type=jnp.float32) + bg_ref[...]
        nz = jnp.dot(hs, wn_ref[...], preferred_element_type=jnp.float32) + bn_ref[...]
        sp = jnp.maximum(nz, 0.0) + jnp.log1p(jnp.exp(-jnp.abs(nz)))
        gate_ref[...] = g
        noisy = g + eps_ref[...] * sp
        noisy_ref[...] = noisy
        noisyT_ref[...] = noisy.T


def _ffn_body(bex_ref, hs_ref, we1_ref, be1_ref, we2_ref, be2_ref, out_ref):
    t = jnp.dot(hs_ref[...], we1_ref[0],
                preferred_element_type=jnp.float32) + be1_ref[0]
    t = 0.5 * t * (1.0 + jax.lax.erf(t * (1.0 / math.sqrt(2.0))))
    out_ref[...] = jnp.dot(t, we2_ref[0],
                           preferred_element_type=jnp.float32) + be2_ref[0]


def _iota16():
    return jax.lax.broadcasted_iota(jnp.int32, (16,), 0)


def _cumsum16(x, tmp_v, lanes):
    zf = jnp.zeros((16,), jnp.float32)
    for k in (1, 2, 4, 8):
        tmp_v[...] = x
        sh = plsc.load_gather(tmp_v, [jnp.maximum(lanes - k, 0)])
        x = x + jnp.where(lanes >= k, sh, zf)
    return x


def _router_body(nlT_ref, hs_ref,
                 pos0_ref, pos1_ref, w0_ref, w1_ref, bex_ref, sorted_ref,
                 nl_v, hs_v, pose_v, wtsl_v, e_v, rank_v, cnt_v, cnts_all_v,
                 offbase_v, csum_v, bex_v, tmp_v, shared, sem_hs, sem_sc):
    sid = lax.axis_index("s")
    base_tok = sid * TPT
    hs_cp = pltpu.make_async_copy(hs_ref.at[pl.ds(base_tok, TPT)], hs_v, sem_hs)
    hs_cp.start()
    pltpu.sync_copy(nlT_ref.at[:, pl.ds(base_tok, TPT)], nl_v)

    lanes = _iota16()
    cnt = [jnp.float32(0)] * E
    for g in range(TPT // 16):
        sl = pl.ds(g * 16, 16)
        vs = [nl_v[e, sl] for e in range(E)]
        m1 = functools.reduce(jnp.maximum, vs)
        e1v = jnp.full((16,), E, jnp.int32)
        for e in range(E):
            e1v = jnp.minimum(e1v, jnp.where(vs[e] == m1, e, E))
        vs2 = [jnp.where(e1v == e, NEG, vs[e]) for e in range(E)]
        m2 = functools.reduce(jnp.maximum, vs2)
        e2v = jnp.full((16,), E, jnp.int32)
        for e in range(E):
            e2v = jnp.minimum(e2v, jnp.where(vs2[e] == m2, e, E))
        a = jnp.exp(m2 - m1)
        wtsl_v[0, sl] = 1.0 / (1.0 + a)
        wtsl_v[1, sl] = a / (1.0 + a)
        rank0 = jnp.zeros((16,), jnp.float32)
        rank1 = jnp.zeros((16,), jnp.float32)
        for e in range(E):
            m0 = e1v == e
            m1b = e2v == e
            mf = jnp.where(jnp.logical_or(m0, m1b), 1.0, 0.0)
            incl = _cumsum16(mf, tmp_v, lanes)
            r = cnt[e] + (incl - mf)
            rank0 = jnp.where(m0, r, rank0)
            rank1 = jnp.where(m1b, r, rank1)
            cnt[e] = cnt[e] + incl[15]
        e_v[0, sl] = e1v
        e_v[1, sl] = e2v
        rank_v[0, sl] = rank0.astype(jnp.int32)
        rank_v[1, sl] = rank1.astype(jnp.int32)

    cv = jnp.zeros((16,), jnp.float32)
    for e in range(E):
        cv = jnp.where(lanes == e, cnt[e], cv)
    cnt_v[pl.ds(0, 16)] = cv
    zf16 = jnp.zeros((16,), jnp.float32)
    for j in range(1, 8):
        cnt_v[pl.ds(j * 16, 16)] = zf16
    pltpu.sync_copy(cnt_v, shared.at[sid])
    plsc.subcore_barrier()
    pltpu.sync_copy(shared, cnts_all_v)

    sidv = jnp.full((16,), sid, jnp.int32)
    basev = jnp.zeros((16,), jnp.float32)
    totv = jnp.zeros((16,), jnp.float32)
    zf = jnp.zeros((16,), jnp.float32)
    for w2 in range(NT1):
        row = cnts_all_v[w2, pl.ds(0, 16)]
        basev = basev + jnp.where(jnp.full((16,), w2, jnp.int32) < sidv, row, zf)
        totv = totv + row
    pc = ((totv.astype(jnp.int32) + (BR - 1)) >> 7) << 7
    pcf = pc.astype(jnp.float32)
    csum = _cumsum16(pcf, tmp_v, lanes)
    off = csum - pcf
    offbase_v[...] = (off + basev).astype(jnp.int32)
    csum_v[...] = csum.astype(jnp.int32)

    for g in range(TPT // 16):
        sl = pl.ds(g * 16, 16)
        pose_v[0, sl] = plsc.load_gather(offbase_v, [e_v[0, sl]]) + rank_v[0, sl]
        pose_v[1, sl] = plsc.load_gather(offbase_v, [e_v[1, sl]]) + rank_v[1, sl]
    pltpu.sync_copy(pose_v.at[0], pos0_ref.at[pl.ds(base_tok, TPT)])
    pltpu.sync_copy(pose_v.at[1], pos1_ref.at[pl.ds(base_tok, TPT)])
    pltpu.sync_copy(wtsl_v.at[0], w0_ref.at[pl.ds(base_tok, TPT)])
    pltpu.sync_copy(wtsl_v.at[1], w1_ref.at[pl.ds(base_tok, TPT)])

    @pl.when(sid == 0)
    def _():
        csumv = csum_v[...]
        for j in range(NB // 16):
            bvec = lanes + j * 16
            acc = jnp.zeros((16,), jnp.int32)
            for e in range(E):
                end_e = csumv[e] >> 7
                acc = acc + jnp.where(bvec >= end_e, 1, 0).astype(jnp.int32)
            bex_v[pl.ds(j * 16, 16)] = jnp.minimum(acc, E - 1)
        pltpu.sync_copy(bex_v, bex_ref)

    hs_cp.wait()
    pltpu.async_copy(hs_v, sorted_ref.at[pose_v.at[0]], sem_sc).wait()
    pltpu.async_copy(hs_v, sorted_ref.at[pose_v.at[1]], sem_sc).wait()


def _combine_body(x_ref, outc_ref, pos0_ref, pos1_ref, w0_ref, w1_ref,
                  final_ref, pos0_v, pos1_v, w0_v, w1_v, a_v, b_v, x_v, f_v,
                  sem):
    wid = lax.axis_index("s") * 2 + lax.axis_index("c")
    base_tok = wid * TPW
    pltpu.sync_copy(pos0_ref.at[pl.ds(base_tok, TPW)], pos0_v)
    pltpu.sync_copy(pos1_ref.at[pl.ds(base_tok, TPW)], pos1_v)
    pltpu.sync_copy(w0_ref.at[pl.ds(base_tok, TPW)], w0_v)
    pltpu.sync_copy(w1_ref.at[pl.ds(base_tok, TPW)], w1_v)
    for jh in range(2):
        tok0 = base_tok + jh * HTPW
        pltpu.async_copy(outc_ref.at[pos0_v.at[pl.ds(jh * HTPW, HTPW)]],
                         a_v, sem).wait()
        pltpu.async_copy(outc_ref.at[pos1_v.at[pl.ds(jh * HTPW, HTPW)]],
                         b_v, sem).wait()
        pltpu.sync_copy(x_ref.at[pl.ds(tok0, HTPW)], x_v)

        def body(t, _):
            ti = jnp.full((16,), jh * HTPW + t, jnp.int32)
            w0 = plsc.load_gather(w0_v, [ti])
            w1 = plsc.load_gather(w1_v, [ti])
            for c in range(C // 16):
                sl = pl.ds(c * 16, 16)
                f_v[t, sl] = x_v[t, sl] + w0 * a_v[t, sl] + w1 * b_v[t, sl]
            return 0

        lax.fori_loop(0, HTPW, body, 0)
        pltpu.sync_copy(f_v, final_ref.at[pl.ds(tok0, HTPW)])


def kernel(hidden_states, ln1_w, ln1_b, ln2_w, ln2_b, Wq, bq, Wk, bk, Wv, bv,
           Wo, bo, Wg, bg, Wn, bn_, We1, be1, We2, be2, noise_eps):
    x2d = hidden_states.reshape(T, C)
    full = lambda r, c: pl.BlockSpec((r, c), lambda *_: (0, 0))

    def _xrow(s):
        return (jnp.where(s < NA, s,
                          jnp.where(s >= NC, s - NC, NA - 1)), 0)

    def _crow(s):
        return (jnp.where(s >= NC, s - NC, 0), 0)

    def _ccol(s):
        return (0, jnp.where(s >= NC, s - NC, 0))

    hs, noisy, gate, noisy_t = pl.pallas_call(
        _fused_body,
        grid=(NC + NA,),
        in_specs=[
            pl.BlockSpec((BT, C), _xrow),
            full(1, C), full(1, C),
            full(C, C), full(1, C), full(C, C), full(1, C),
            full(C, C), full(1, C),
            pl.BlockSpec((BT, E), _crow),
            full(C, C), full(1, C), full(1, C), full(1, C),
            full(C, E), full(1, E), full(C, E), full(1, E),
        ],
        out_specs=[
            pl.BlockSpec((BT, C), _crow),
            pl.BlockSpec((BT, E), _crow),
            pl.BlockSpec((BT, E), _crow),
            pl.BlockSpec((E, BT), _ccol),
        ],
        out_shape=[
            jax.ShapeDtypeStruct((T, C), jnp.float32),
            jax.ShapeDtypeStruct((T, E), jnp.float32),
            jax.ShapeDtypeStruct((T, E), jnp.float32),
            jax.ShapeDtypeStruct((E, T), jnp.float32),
        ],
        scratch_shapes=[
            pltpu.VMEM((H2, T, 2 * HD), jnp.float32),
            pltpu.VMEM((H2, T, 2 * HD), jnp.float32),
            pltpu.VMEM((H2, T, 2 * HD), jnp.float32),
            pltpu.VMEM((H2, T, 2 * HD), jnp.float32),
        ],
    )(x2d, ln1_w.reshape(1, C), ln1_b.reshape(1, C), Wq, bq.reshape(1, C),
      Wk, bk.reshape(1, C), Wv, bv.reshape(1, C), noise_eps,
      Wo, bo.reshape(1, C), ln2_w.reshape(1, C), ln2_b.reshape(1, C),
      Wg, bg.reshape(1, E), Wn, bn_.reshape(1, E))

    router = functools.partial(
        pl.kernel,
        out_type=[
            jax.ShapeDtypeStruct((T,), jnp.int32),
            jax.ShapeDtypeStruct((T,), jnp.int32),
            jax.ShapeDtypeStruct((T,), jnp.float32),
            jax.ShapeDtypeStruct((T,), jnp.float32),
            jax.ShapeDtypeStruct((NB,), jnp.int32),
            jax.ShapeDtypeStruct((NR, C), jnp.float32),
        ],
        mesh=plsc.VectorSubcoreMesh(core_axis_name="c", subcore_axis_name="s",
                                    num_cores=1),
        compiler_params=pltpu.CompilerParams(needs_layout_passes=False),
        scratch_types=[
            pltpu.VMEM((E, TPT), jnp.float32),
            pltpu.VMEM((TPT, C), jnp.float32),
            pltpu.VMEM((2, TPT), jnp.int32),
            pltpu.VMEM((2, TPT), jnp.float32),
            pltpu.VMEM((2, TPT), jnp.int32),
            pltpu.VMEM((2, TPT), jnp.int32),
            pltpu.VMEM((128,), jnp.float32),
            pltpu.VMEM((NT1, 128), jnp.float32),
            pltpu.VMEM((16,), jnp.int32),
            pltpu.VMEM((16,), jnp.int32),
            pltpu.VMEM((NB,), jnp.int32),
            pltpu.VMEM((16,), jnp.float32),
            pltpu.VMEM_SHARED((NT1, 128), jnp.float32),
            pltpu.SemaphoreType.DMA,
            pltpu.SemaphoreType.DMA,
        ],
    )(_router_body)
    pos0, pos1, w0, w1, bex, sorted_rows = router(noisy_t, hs)

    grid_spec = pltpu.PrefetchScalarGridSpec(
        num_scalar_prefetch=1,
        grid=(NB,),
        in_specs=[
            pl.BlockSpec((BR, C), lambda g, b: (g, 0)),
            pl.BlockSpec((1, C, FF), lambda g, b: (b[g], 0, 0)),
            pl.BlockSpec((1, 1, FF), lambda g, b: (b[g], 0, 0)),
            pl.BlockSpec((1, FF, C), lambda g, b: (b[g], 0, 0)),
            pl.BlockSpec((1, 1, C), lambda g, b: (b[g], 0, 0)),
        ],
        out_specs=pl.BlockSpec((BR, C), lambda g, b: (g, 0)),
    )
    outc = pl.pallas_call(
        _ffn_body,
        grid_spec=grid_spec,
        out_shape=jax.ShapeDtypeStruct((NR, C), jnp.float32),
    )(bex, sorted_rows, We1, be1.reshape(E, 1, FF),
      We2, be2.reshape(E, 1, C))

    combine = functools.partial(
        pl.kernel,
        out_type=jax.ShapeDtypeStruct((T, C), jnp.float32),
        mesh=plsc.VectorSubcoreMesh(core_axis_name="c", subcore_axis_name="s"),
        compiler_params=pltpu.CompilerParams(needs_layout_passes=False),
        scratch_types=[
            pltpu.VMEM((TPW,), jnp.int32),
            pltpu.VMEM((TPW,), jnp.int32),
            pltpu.VMEM((TPW,), jnp.float32),
            pltpu.VMEM((TPW,), jnp.float32),
            pltpu.VMEM((HTPW, C), jnp.float32),
            pltpu.VMEM((HTPW, C), jnp.float32),
            pltpu.VMEM((HTPW, C), jnp.float32),
            pltpu.VMEM((HTPW, C), jnp.float32),
            pltpu.SemaphoreType.DMA,
        ],
    )(_combine_body)
    final = combine(x2d, outc, pos0, pos1, w0, w1)

    return (final.reshape(B, T, C), noisy, gate)


# speedup vs baseline: 1.0415x; 1.0050x over previous
import functools
import math

import jax
import jax.numpy as jnp
from jax import lax
from jax.experimental import pallas as pl
from jax.experimental.pallas import tpu as pltpu
from jax.experimental.pallas import tpu_sc as plsc

B, T, C, H, E, K, FF = 1, 2048, 768, 12, 16, 2, 3072
HD = C // H
BT = 256
BR = 128
NB = T * K // BR + E
NR = NB * BR
NEG = -1e30
NT1 = 16
TPT = T // NT1
NW = 32
TPW = T // NW
HTPW = TPW // 2


H2 = H // 2
NA = T // BT
NBS = H2 * (T // BT)
NC = NA + NBS


def _fused_body(x_ref, ln1w_ref, ln1b_ref, wq_ref, bq_ref, wk_ref, bk_ref,
                wv_ref, bv_ref, eps_ref, wo_ref, bo_ref, ln2w_ref, ln2b_ref,
                wg_ref, bg_ref, wn_ref, bn_ref,
                hs_ref, noisy_ref, gate_ref, noisyT_ref,
                qs, ks, vs, ys):
    s = pl.program_id(0)

    @pl.when(s < NA)
    def _():
        i = s
        x = x_ref[...]
        mu = jnp.mean(x, -1, keepdims=True)
        xc = x - mu
        var = jnp.mean(xc * xc, -1, keepdims=True)
        xn = xc * jax.lax.rsqrt(var + 1e-5) * ln1w_ref[...] + ln1b_ref[...]
        q = jnp.dot(xn, wq_ref[...], preferred_element_type=jnp.float32) + bq_ref[...]
        k = jnp.dot(xn, wk_ref[...], preferred_element_type=jnp.float32) + bk_ref[...]
        v = jnp.dot(xn, wv_ref[...], preferred_element_type=jnp.float32) + bv_ref[...]
        rsl = pl.ds(i * BT, BT)
        for h2 in range(H2):
            csl = slice(h2 * 2 * HD, (h2 + 1) * 2 * HD)
            qs[h2, rsl, :] = q[:, csl]
            ks[h2, rsl, :] = k[:, csl]
            vs[h2, rsl, :] = v[:, csl]

    @pl.when((s >= NA) & (s < NC))
    def _():
        idx = s - NA
        h2d = idx // (T // BT)
        i = idx % (T // BT)
        q2 = qs[h2d, pl.ds(i * BT, BT), :]
        k2 = ks[h2d]
        v2 = vs[h2d]
        rows = jax.lax.broadcasted_iota(jnp.int32, (BT, T), 0) + i * BT
        cols = jax.lax.broadcasted_iota(jnp.int32, (BT, T), 1)
        causal = cols <= rows
        for hh in range(2):
            sl = slice(hh * HD, (hh + 1) * HD)
            sc = jax.lax.dot_general(q2[:, sl], k2[:, sl],
                                     (((1,), (1,)), ((), ())),
                                     preferred_element_type=jnp.float32)
            sc = sc * (1.0 / math.sqrt(HD))
            sc = jnp.where(causal, sc, NEG)
            m = jnp.max(sc, -1, keepdims=True)
            p = jnp.exp(sc - m)
            p = p / jnp.sum(p, -1, keepdims=True)
            ys[h2d, pl.ds(i * BT, BT), sl] = jnp.dot(
                p, v2[:, sl], preferred_element_type=jnp.float32)

    @pl.when(s >= NC)
    def _():
        i = s - NC
        y = jnp.concatenate(
            [ys[h2, pl.ds(i * BT, BT), :] for h2 in range(H2)], axis=1)
        h = x_ref[...] + jnp.dot(y, wo_ref[...],
                                 preferred_element_type=jnp.float32) + bo_ref[...]
        mu = jnp.mean(h, -1, keepdims=True)
        hc = h - mu
        var = jnp.mean(hc * hc, -1, keepdims=True)
        hs = hc * jax.lax.rsqrt(var + 1e-5) * ln2w_ref[...] + ln2b_ref[...]
        hs_ref[...] = hs
        g = jnp.dot(hs, wg_ref[...], preferred_element_type=jnp.float32) + bg_ref[...]
        nz = jnp.dot(hs, wn_ref[...], preferred_element_type=jnp.float32) + bn_ref[...]
        sp = jnp.maximum(nz, 0.0) + jnp.log1p(jnp.exp(-jnp.abs(nz)))
        gate_ref[...] = g
        noisy = g + eps_ref[...] * sp
        noisy_ref[...] = noisy
        noisyT_ref[...] = noisy.T


def _ffn_body(bex_ref, hs_ref, we1_ref, be1_ref, we2_ref, be2_ref, out_ref):
    t = jnp.dot(hs_ref[...], we1_ref[0],
                preferred_element_type=jnp.float32) + be1_ref[0]
    t = 0.5 * t * (1.0 + jax.lax.erf(t * (1.0 / math.sqrt(2.0))))
    out_ref[...] = jnp.dot(t, we2_ref[0],
                           preferred_element_type=jnp.float32) + be2_ref[0]


def _iota16():
    return jax.lax.broadcasted_iota(jnp.int32, (16,), 0)


def _cumsum16(x, tmp_v, lanes):
    zf = jnp.zeros((16,), jnp.float32)
    for k in (1, 2, 4, 8):
        tmp_v[...] = x
        sh = plsc.load_gather(tmp_v, [jnp.maximum(lanes - k, 0)])
        x = x + jnp.where(lanes >= k, sh, zf)
    return x


def _router_body(nlT_ref, hs_ref,
                 pos0_ref, pos1_ref, w0_ref, w1_ref, bex_ref, sorted_ref,
                 nl_v, hs_v, pose_v, wtsl_v, e_v, rank_v, cnt_v, cnts_all_v,
                 offbase_v, csum_v, bex_v, tmp_v, shared, sem_hs, sem_sc):
    sid = lax.axis_index("s")
    base_tok = sid * TPT
    hs_cp = pltpu.make_async_copy(hs_ref.at[pl.ds(base_tok, TPT)], hs_v, sem_hs)
    hs_cp.start()
    pltpu.sync_copy(nlT_ref.at[:, pl.ds(base_tok, TPT)], nl_v)

    lanes = _iota16()
    cnt = [jnp.float32(0)] * E
    for g in range(TPT // 16):
        sl = pl.ds(g * 16, 16)
        vs = [nl_v[e, sl] for e in range(E)]
        m1 = functools.reduce(jnp.maximum, vs)
        e1v = jnp.full((16,), E, jnp.int32)
        for e in range(E):
            e1v = jnp.minimum(e1v, jnp.where(vs[e] == m1, e, E))
        vs2 = [jnp.where(e1v == e, NEG, vs[e]) for e in range(E)]
        m2 = functools.reduce(jnp.maximum, vs2)
        e2v = jnp.full((16,), E, jnp.int32)
        for e in range(E):
            e2v = jnp.minimum(e2v, jnp.where(vs2[e] == m2, e, E))
        a = jnp.exp(m2 - m1)
        wtsl_v[0, sl] = 1.0 / (1.0 + a)
        wtsl_v[1, sl] = a / (1.0 + a)
        rank0 = jnp.zeros((16,), jnp.float32)
        rank1 = jnp.zeros((16,), jnp.float32)
        for e in range(E):
            m0 = e1v == e
            m1b = e2v == e
            mf = jnp.where(jnp.logical_or(m0, m1b), 1.0, 0.0)
            incl = _cumsum16(mf, tmp_v, lanes)
            r = cnt[e] + (incl - mf)
            rank0 = jnp.where(m0, r, rank0)
            rank1 = jnp.where(m1b, r, rank1)
            cnt[e] = cnt[e] + incl[15]
        e_v[0, sl] = e1v
        e_v[1, sl] = e2v
        rank_v[0, sl] = rank0.astype(jnp.int32)
        rank_v[1, sl] = rank1.astype(jnp.int32)

    cv = jnp.zeros((16,), jnp.float32)
    for e in range(E):
        cv = jnp.where(lanes == e, cnt[e], cv)
    cnt_v[pl.ds(0, 16)] = cv
    zf16 = jnp.zeros((16,), jnp.float32)
    for j in range(1, 8):
        cnt_v[pl.ds(j * 16, 16)] = zf16
    pltpu.sync_copy(cnt_v, shared.at[sid])
    plsc.subcore_barrier()
    pltpu.sync_copy(shared, cnts_all_v)

    sidv = jnp.full((16,), sid, jnp.int32)
    basev = jnp.zeros((16,), jnp.float32)
    totv = jnp.zeros((16,), jnp.float32)
    zf = jnp.zeros((16,), jnp.float32)
    for w2 in range(NT1):
        row = cnts_all_v[w2, pl.ds(0, 16)]
        basev = basev + jnp.where(jnp.full((16,), w2, jnp.int32) < sidv, row, zf)
        totv = totv + row
    pc = ((totv.astype(jnp.int32) + (BR - 1)) >> 7) << 7
    pcf = pc.astype(jnp.float32)
    csum = _cumsum16(pcf, tmp_v, lanes)
    off = csum - pcf
    offbase_v[...] = (off + basev).astype(jnp.int32)
    csum_v[...] = csum.astype(jnp.int32)

    for g in range(TPT // 16):
        sl = pl.ds(g * 16, 16)
        pose_v[0, sl] = plsc.load_gather(offbase_v, [e_v[0, sl]]) + rank_v[0, sl]
        pose_v[1, sl] = plsc.load_gather(offbase_v, [e_v[1, sl]]) + rank_v[1, sl]
    pltpu.sync_copy(pose_v.at[0], pos0_ref.at[pl.ds(base_tok, TPT)])
    pltpu.sync_copy(pose_v.at[1], pos1_ref.at[pl.ds(base_tok, TPT)])
    pltpu.sync_copy(wtsl_v.at[0], w0_ref.at[pl.ds(base_tok, TPT)])
    pltpu.sync_copy(wtsl_v.at[1], w1_ref.at[pl.ds(base_tok, TPT)])

    @pl.when(sid == 0)
    def _():
        csumv = csum_v[...]
        for j in range(NB // 16):
            bvec = lanes + j * 16
            acc = jnp.zeros((16,), jnp.int32)
            for e in range(E):
                end_e = csumv[e] >> 7
                acc = acc + jnp.where(bvec >= end_e, 1, 0).astype(jnp.int32)
            bex_v[pl.ds(j * 16, 16)] = jnp.minimum(acc, E - 1)
        pltpu.sync_copy(bex_v, bex_ref)

    hs_cp.wait()
    c0 = pltpu.make_async_copy(hs_v, sorted_ref.at[pose_v.at[0]], sem_sc)
    c1 = pltpu.make_async_copy(hs_v, sorted_ref.at[pose_v.at[1]], sem_sc)
    c0.start()
    c1.start()
    c0.wait()
    c1.wait()


def _combine_body(x_ref, outc_ref, pos0_ref, pos1_ref, w0_ref, w1_ref,
                  final_ref, pos0_v, pos1_v, w0_v, w1_v, a_v, b_v, x_v, f_v,
                  sem):
    wid = lax.axis_index("s") * 2 + lax.axis_index("c")
    base_tok = wid * TPW
    pltpu.sync_copy(pos0_ref.at[pl.ds(base_tok, TPW)], pos0_v)
    pltpu.sync_copy(pos1_ref.at[pl.ds(base_tok, TPW)], pos1_v)
    pltpu.sync_copy(w0_ref.at[pl.ds(base_tok, TPW)], w0_v)
    pltpu.sync_copy(w1_ref.at[pl.ds(base_tok, TPW)], w1_v)
    for jh in range(2):
        tok0 = base_tok + jh * HTPW
        ca = pltpu.make_async_copy(
            outc_ref.at[pos0_v.at[pl.ds(jh * HTPW, HTPW)]], a_v, sem)
        cb = pltpu.make_async_copy(
            outc_ref.at[pos1_v.at[pl.ds(jh * HTPW, HTPW)]], b_v, sem)
        ca.start()
        cb.start()
        pltpu.sync_copy(x_ref.at[pl.ds(tok0, HTPW)], x_v)
        ca.wait()
        cb.wait()

        def body(t, _):
            ti = jnp.full((16,), jh * HTPW + t, jnp.int32)
            w0 = plsc.load_gather(w0_v, [ti])
            w1 = plsc.load_gather(w1_v, [ti])
            for c in range(C // 16):
                sl = pl.ds(c * 16, 16)
                f_v[t, sl] = x_v[t, sl] + w0 * a_v[t, sl] + w1 * b_v[t, sl]
            return 0

        lax.fori_loop(0, HTPW, body, 0)
        pltpu.sync_copy(f_v, final_ref.at[pl.ds(tok0, HTPW)])


def kernel(hidden_states, ln1_w, ln1_b, ln2_w, ln2_b, Wq, bq, Wk, bk, Wv, bv,
           Wo, bo, Wg, bg, Wn, bn_, We1, be1, We2, be2, noise_eps):
    x2d = hidden_states.reshape(T, C)
    full = lambda r, c: pl.BlockSpec((r, c), lambda *_: (0, 0))

    def _xrow(s):
        return (jnp.where(s < NA, s,
                          jnp.where(s >= NC, s - NC, NA - 1)), 0)

    def _crow(s):
        return (jnp.where(s >= NC, s - NC, 0), 0)

    def _ccol(s):
        return (0, jnp.where(s >= NC, s - NC, 0))

    hs, noisy, gate, noisy_t = pl.pallas_call(
        _fused_body,
        grid=(NC + NA,),
        in_specs=[
            pl.BlockSpec((BT, C), _xrow),
            full(1, C), full(1, C),
            full(C, C), full(1, C), full(C, C), full(1, C),
            full(C, C), full(1, C),
            pl.BlockSpec((BT, E), _crow),
            full(C, C), full(1, C), full(1, C), full(1, C),
            full(C, E), full(1, E), full(C, E), full(1, E),
        ],
        out_specs=[
            pl.BlockSpec((BT, C), _crow),
            pl.BlockSpec((BT, E), _crow),
            pl.BlockSpec((BT, E), _crow),
            pl.BlockSpec((E, BT), _ccol),
        ],
        out_shape=[
            jax.ShapeDtypeStruct((T, C), jnp.float32),
            jax.ShapeDtypeStruct((T, E), jnp.float32),
            jax.ShapeDtypeStruct((T, E), jnp.float32),
            jax.ShapeDtypeStruct((E, T), jnp.float32),
        ],
        scratch_shapes=[
            pltpu.VMEM((H2, T, 2 * HD), jnp.float32),
            pltpu.VMEM((H2, T, 2 * HD), jnp.float32),
            pltpu.VMEM((H2, T, 2 * HD), jnp.float32),
            pltpu.VMEM((H2, T, 2 * HD), jnp.float32),
        ],
    )(x2d, ln1_w.reshape(1, C), ln1_b.reshape(1, C), Wq, bq.reshape(1, C),
      Wk, bk.reshape(1, C), Wv, bv.reshape(1, C), noise_eps,
      Wo, bo.reshape(1, C), ln2_w.reshape(1, C), ln2_b.reshape(1, C),
      Wg, bg.reshape(1, E), Wn, bn_.reshape(1, E))

    router = functools.partial(
        pl.kernel,
        out_type=[
            jax.ShapeDtypeStruct((T,), jnp.int32),
            jax.ShapeDtypeStruct((T,), jnp.int32),
            jax.ShapeDtypeStruct((T,), jnp.float32),
            jax.ShapeDtypeStruct((T,), jnp.float32),
            jax.ShapeDtypeStruct((NB,), jnp.int32),
            jax.ShapeDtypeStruct((NR, C), jnp.float32),
        ],
        mesh=plsc.VectorSubcoreMesh(core_axis_name="c", subcore_axis_name="s",
                                    num_cores=1),
        compiler_params=pltpu.CompilerParams(needs_layout_passes=False),
        scratch_types=[
            pltpu.VMEM((E, TPT), jnp.float32),
            pltpu.VMEM((TPT, C), jnp.float32),
            pltpu.VMEM((2, TPT), jnp.int32),
            pltpu.VMEM((2, TPT), jnp.float32),
            pltpu.VMEM((2, TPT), jnp.int32),
            pltpu.VMEM((2, TPT), jnp.int32),
            pltpu.VMEM((128,), jnp.float32),
            pltpu.VMEM((NT1, 128), jnp.float32),
            pltpu.VMEM((16,), jnp.int32),
            pltpu.VMEM((16,), jnp.int32),
            pltpu.VMEM((NB,), jnp.int32),
            pltpu.VMEM((16,), jnp.float32),
            pltpu.VMEM_SHARED((NT1, 128), jnp.float32),
            pltpu.SemaphoreType.DMA,
            pltpu.SemaphoreType.DMA,
        ],
    )(_router_body)
    pos0, pos1, w0, w1, bex, sorted_rows = router(noisy_t, hs)

    grid_spec = pltpu.PrefetchScalarGridSpec(
        num_scalar_prefetch=1,
        grid=(NB,),
        in_specs=[
            pl.BlockSpec((BR, C), lambda g, b: (g, 0)),
            pl.BlockSpec((1, C, FF), lambda g, b: (b[g], 0, 0)),
            pl.BlockSpec((1, 1, FF), lambda g, b: (b[g], 0, 0)),
            pl.BlockSpec((1, FF, C), lambda g, b: (b[g], 0, 0)),
            pl.BlockSpec((1, 1, C), lambda g, b: (b[g], 0, 0)),
        ],
        out_specs=pl.BlockSpec((BR, C), lambda g, b: (g, 0)),
    )
    outc = pl.pallas_call(
        _ffn_body,
        grid_spec=grid_spec,
        out_shape=jax.ShapeDtypeStruct((NR, C), jnp.float32),
    )(bex, sorted_rows, We1, be1.reshape(E, 1, FF),
      We2, be2.reshape(E, 1, C))

    combine = functools.partial(
        pl.kernel,
        out_type=jax.ShapeDtypeStruct((T, C), jnp.float32),
        mesh=plsc.VectorSubcoreMesh(core_axis_name="c", subcore_axis_name="s"),
        compiler_params=pltpu.CompilerParams(needs_layout_passes=False),
        scratch_types=[
            pltpu.VMEM((TPW,), jnp.int32),
            pltpu.VMEM((TPW,), jnp.int32),
            pltpu.VMEM((TPW,), jnp.float32),
            pltpu.VMEM((TPW,), jnp.float32),
            pltpu.VMEM((HTPW, C), jnp.float32),
            pltpu.VMEM((HTPW, C), jnp.float32),
            pltpu.VMEM((HTPW, C), jnp.float32),
            pltpu.VMEM((HTPW, C), jnp.float32),
            pltpu.SemaphoreType.DMA,
        ],
    )(_combine_body)
    final = combine(x2d, outc, pos0, pos1, w0, w1)

    return (final.reshape(B, T, C), noisy, gate)


# speedup vs baseline: 1.0910x; 1.0476x over previous
import functools
import math

import jax
import jax.numpy as jnp
from jax import lax
from jax.experimental import pallas as pl
from jax.experimental.pallas import tpu as pltpu
from jax.experimental.pallas import tpu_sc as plsc

B, T, C, H, E, K, FF = 1, 2048, 768, 12, 16, 2, 3072
HD = C // H
BT = 512
BR = 128
NB = T * K // BR + E
NR = NB * BR
NEG = -1e30
NT1 = 16
TPT = T // NT1
NW = 32
TPW = T // NW
HTPW = TPW // 2


H2 = H // 2
NA = T // BT
NBS = H2 * (T // BT)
NC = NA + NBS


def _fused_body(x_ref, ln1w_ref, ln1b_ref, wq_ref, bq_ref, wk_ref, bk_ref,
                wv_ref, bv_ref, eps_ref, wo_ref, bo_ref, ln2w_ref, ln2b_ref,
                wg_ref, bg_ref, wn_ref, bn_ref,
                hs_ref, noisy_ref, gate_ref, noisyT_ref,
                qs, ks, vs, ys):
    s = pl.program_id(0)

    @pl.when(s < NA)
    def _():
        i = s
        x = x_ref[...]
        mu = jnp.mean(x, -1, keepdims=True)
        xc = x - mu
        var = jnp.mean(xc * xc, -1, keepdims=True)
        xn = xc / jnp.sqrt(var + 1e-5) * ln1w_ref[...] + ln1b_ref[...]
        q = jnp.dot(xn, wq_ref[...], preferred_element_type=jnp.float32) + bq_ref[...]
        k = jnp.dot(xn, wk_ref[...], preferred_element_type=jnp.float32) + bk_ref[...]
        v = jnp.dot(xn, wv_ref[...], preferred_element_type=jnp.float32) + bv_ref[...]
        rsl = pl.ds(i * BT, BT)
        for h2 in range(H2):
            csl = slice(h2 * 2 * HD, (h2 + 1) * 2 * HD)
            qs[h2, rsl, :] = q[:, csl]
            ks[h2, rsl, :] = k[:, csl]
            vs[h2, rsl, :] = v[:, csl]

    @pl.when((s >= NA) & (s < NC))
    def _():
        idx = s - NA
        h2d = idx // (T // BT)
        i = idx % (T // BT)
        q2 = qs[h2d, pl.ds(i * BT, BT), :]
        k2 = ks[h2d]
        v2 = vs[h2d]
        rows = jax.lax.broadcasted_iota(jnp.int32, (BT, T), 0) + i * BT
        cols = jax.lax.broadcasted_iota(jnp.int32, (BT, T), 1)
        causal = cols <= rows
        for hh in range(2):
            sl = slice(hh * HD, (hh + 1) * HD)
            sc = jax.lax.dot_general(q2[:, sl], k2[:, sl],
                                     (((1,), (1,)), ((), ())),
                                     preferred_element_type=jnp.float32)
            sc = sc * (1.0 / math.sqrt(HD))
            sc = jnp.where(causal, sc, NEG)
            m = jnp.max(sc, -1, keepdims=True)
            p = jnp.exp(sc - m)
            p = p / jnp.sum(p, -1, keepdims=True)
            ys[h2d, pl.ds(i * BT, BT), sl] = jnp.dot(
                p, v2[:, sl], preferred_element_type=jnp.float32)

    @pl.when(s >= NC)
    def _():
        i = s - NC
        y = jnp.concatenate(
            [ys[h2, pl.ds(i * BT, BT), :] for h2 in range(H2)], axis=1)
        h = x_ref[...] + (jnp.dot(y, wo_ref[...],
                                  preferred_element_type=jnp.float32)
                          + bo_ref[...])
        mu = jnp.mean(h, -1, keepdims=True)
        hc = h - mu
        var = jnp.mean(hc * hc, -1, keepdims=True)
        hs = hc / jnp.sqrt(var + 1e-5) * ln2w_ref[...] + ln2b_ref[...]
        hs_ref[...] = hs
        g = jnp.dot(hs, wg_ref[...], preferred_element_type=jnp.float32) + bg_ref[...]
        nz = jnp.dot(hs, wn_ref[...], preferred_element_type=jnp.float32) + bn_ref[...]
        sp = jnp.maximum(nz, 0.0) + jnp.log1p(jnp.exp(-jnp.abs(nz)))
        gate_ref[...] = g
        noisy = g + eps_ref[...] * sp
        noisy_ref[...] = noisy
        noisyT_ref[...] = noisy.T


def _ffn_body(bex_ref, hs_ref, we1_ref, be1_ref, we2_ref, be2_ref, out_ref):
    t = jnp.dot(hs_ref[...], we1_ref[0],
                preferred_element_type=jnp.float32) + be1_ref[0]
    t = 0.5 * t * (1.0 + jax.lax.erf(t * (1.0 / math.sqrt(2.0))))
    out_ref[...] = jnp.dot(t, we2_ref[0],
                           preferred_element_type=jnp.float32) + be2_ref[0]


def _iota16():
    return jax.lax.broadcasted_iota(jnp.int32, (16,), 0)


def _cumsum16(x, tmp_v, lanes):
    zf = jnp.zeros((16,), jnp.float32)
    for k in (1, 2, 4, 8):
        tmp_v[...] = x
        sh = plsc.load_gather(tmp_v, [jnp.maximum(lanes - k, 0)])
        x = x + jnp.where(lanes >= k, sh, zf)
    return x


def _router_body(nlT_ref, hs_ref,
                 pos0_ref, pos1_ref, w0_ref, w1_ref, bex_ref, sorted_ref,
                 nl_v, hs_v, pose_v, wtsl_v, e_v, rank_v, cnt_v, cnts_all_v,
                 offbase_v, csum_v, bex_v, tmp_v, shared, sem_hs, sem_sc):
    sid = lax.axis_index("s")
    base_tok = sid * TPT
    hs_cp = pltpu.make_async_copy(hs_ref.at[pl.ds(base_tok, TPT)], hs_v, sem_hs)
    hs_cp.start()
    pltpu.sync_copy(nlT_ref.at[:, pl.ds(base_tok, TPT)], nl_v)

    lanes = _iota16()
    cnt = [jnp.float32(0)] * E
    for g in range(TPT // 16):
        sl = pl.ds(g * 16, 16)
        vs = [nl_v[e, sl] for e in range(E)]
        m1 = functools.reduce(jnp.maximum, vs)
        e1v = jnp.full((16,), E, jnp.int32)
        for e in range(E):
            e1v = jnp.minimum(e1v, jnp.where(vs[e] == m1, e, E))
        vs2 = [jnp.where(e1v == e, NEG, vs[e]) for e in range(E)]
        m2 = functools.reduce(jnp.maximum, vs2)
        e2v = jnp.full((16,), E, jnp.int32)
        for e in range(E):
            e2v = jnp.minimum(e2v, jnp.where(vs2[e] == m2, e, E))
        a = jnp.exp(m2 - m1)
        wtsl_v[0, sl] = 1.0 / (1.0 + a)
        wtsl_v[1, sl] = a / (1.0 + a)
        rank0 = jnp.zeros((16,), jnp.float32)
        rank1 = jnp.zeros((16,), jnp.float32)
        for e in range(E):
            m0 = e1v == e
            m1b = e2v == e
            mf = jnp.where(jnp.logical_or(m0, m1b), 1.0, 0.0)
            incl = _cumsum16(mf, tmp_v, lanes)
            r = cnt[e] + (incl - mf)
            rank0 = jnp.where(m0, r, rank0)
            rank1 = jnp.where(m1b, r, rank1)
            cnt[e] = cnt[e] + incl[15]
        e_v[0, sl] = e1v
        e_v[1, sl] = e2v
        rank_v[0, sl] = rank0.astype(jnp.int32)
        rank_v[1, sl] = rank1.astype(jnp.int32)

    cv = jnp.zeros((16,), jnp.float32)
    for e in range(E):
        cv = jnp.where(lanes == e, cnt[e], cv)
    cnt_v[pl.ds(0, 16)] = cv
    zf16 = jnp.zeros((16,), jnp.float32)
    for j in range(1, 8):
        cnt_v[pl.ds(j * 16, 16)] = zf16
    pltpu.sync_copy(cnt_v, shared.at[sid])
    plsc.subcore_barrier()
    pltpu.sync_copy(shared, cnts_all_v)

    sidv = jnp.full((16,), sid, jnp.int32)
    basev = jnp.zeros((16,), jnp.float32)
    totv = jnp.zeros((16,), jnp.float32)
    zf = jnp.zeros((16,), jnp.float32)
    for w2 in range(NT1):
        row = cnts_all_v[w2, pl.ds(0, 16)]
        basev = basev + jnp.where(jnp.full((16,), w2, jnp.int32) < sidv, row, zf)
        totv = totv + row
    pc = ((totv.astype(jnp.int32) + (BR - 1)) >> 7) << 7
    pcf = pc.astype(jnp.float32)
    csum = _cumsum16(pcf, tmp_v, lanes)
    off = csum - pcf
    offbase_v[...] = (off + basev).astype(jnp.int32)
    csum_v[...] = csum.astype(jnp.int32)

    for g in range(TPT // 16):
        sl = pl.ds(g * 16, 16)
        pose_v[0, sl] = plsc.load_gather(offbase_v, [e_v[0, sl]]) + rank_v[0, sl]
        pose_v[1, sl] = plsc.load_gather(offbase_v, [e_v[1, sl]]) + rank_v[1, sl]
    pltpu.sync_copy(pose_v.at[0], pos0_ref.at[pl.ds(base_tok, TPT)])
    pltpu.sync_copy(pose_v.at[1], pos1_ref.at[pl.ds(base_tok, TPT)])
    pltpu.sync_copy(wtsl_v.at[0], w0_ref.at[pl.ds(base_tok, TPT)])
    pltpu.sync_copy(wtsl_v.at[1], w1_ref.at[pl.ds(base_tok, TPT)])

    @pl.when(sid == 0)
    def _():
        csumv = csum_v[...]
        for j in range(NB // 16):
            bvec = lanes + j * 16
            acc = jnp.zeros((16,), jnp.int32)
            for e in range(E):
                end_e = csumv[e] >> 7
                acc = acc + jnp.where(bvec >= end_e, 1, 0).astype(jnp.int32)
            bex_v[pl.ds(j * 16, 16)] = jnp.minimum(acc, E - 1)
        pltpu.sync_copy(bex_v, bex_ref)

    hs_cp.wait()
    c0 = pltpu.make_async_copy(hs_v, sorted_ref.at[pose_v.at[0]], sem_sc)
    c1 = pltpu.make_async_copy(hs_v, sorted_ref.at[pose_v.at[1]], sem_sc)
    c0.start()
    c1.start()
    c0.wait()
    c1.wait()


def _combine_body(x_ref, outc_ref, pos0_ref, pos1_ref, w0_ref, w1_ref,
                  final_ref, pos0_v, pos1_v, w0_v, w1_v, a_v, b_v, x_v, f_v,
                  sem):
    wid = lax.axis_index("s") * 2 + lax.axis_index("c")
    base_tok = wid * TPW
    pltpu.sync_copy(pos0_ref.at[pl.ds(base_tok, TPW)], pos0_v)
    pltpu.sync_copy(pos1_ref.at[pl.ds(base_tok, TPW)], pos1_v)
    pltpu.sync_copy(w0_ref.at[pl.ds(base_tok, TPW)], w0_v)
    pltpu.sync_copy(w1_ref.at[pl.ds(base_tok, TPW)], w1_v)
    for jh in range(2):
        tok0 = base_tok + jh * HTPW
        ca = pltpu.make_async_copy(
            outc_ref.at[pos0_v.at[pl.ds(jh * HTPW, HTPW)]], a_v, sem)
        cb = pltpu.make_async_copy(
            outc_ref.at[pos1_v.at[pl.ds(jh * HTPW, HTPW)]], b_v, sem)
        ca.start()
        cb.start()
        pltpu.sync_copy(x_ref.at[pl.ds(tok0, HTPW)], x_v)
        ca.wait()
        cb.wait()

        def body(t, _):
            ti = jnp.full((16,), jh * HTPW + t, jnp.int32)
            w0 = plsc.load_gather(w0_v, [ti])
            w1 = plsc.load_gather(w1_v, [ti])
            for c in range(C // 16):
                sl = pl.ds(c * 16, 16)
                f_v[t, sl] = x_v[t, sl] + w0 * a_v[t, sl] + w1 * b_v[t, sl]
            return 0

        lax.fori_loop(0, HTPW, body, 0)
        pltpu.sync_copy(f_v, final_ref.at[pl.ds(tok0, HTPW)])


def kernel(hidden_states, ln1_w, ln1_b, ln2_w, ln2_b, Wq, bq, Wk, bk, Wv, bv,
           Wo, bo, Wg, bg, Wn, bn_, We1, be1, We2, be2, noise_eps):
    x2d = hidden_states.reshape(T, C)
    full = lambda r, c: pl.BlockSpec((r, c), lambda *_: (0, 0))

    def _xrow(s):
        return (jnp.where(s < NA, s,
                          jnp.where(s >= NC, s - NC, NA - 1)), 0)

    def _crow(s):
        return (jnp.where(s >= NC, s - NC, 0), 0)

    def _ccol(s):
        return (0, jnp.where(s >= NC, s - NC, 0))

    hs, noisy, gate, noisy_t = pl.pallas_call(
        _fused_body,
        grid=(NC + NA,),
        in_specs=[
            pl.BlockSpec((BT, C), _xrow),
            full(1, C), full(1, C),
            full(C, C), full(1, C), full(C, C), full(1, C),
            full(C, C), full(1, C),
            pl.BlockSpec((BT, E), _crow),
            full(C, C), full(1, C), full(1, C), full(1, C),
            full(C, E), full(1, E), full(C, E), full(1, E),
        ],
        out_specs=[
            pl.BlockSpec((BT, C), _crow),
            pl.BlockSpec((BT, E), _crow),
            pl.BlockSpec((BT, E), _crow),
            pl.BlockSpec((E, BT), _ccol),
        ],
        out_shape=[
            jax.ShapeDtypeStruct((T, C), jnp.float32),
            jax.ShapeDtypeStruct((T, E), jnp.float32),
            jax.ShapeDtypeStruct((T, E), jnp.float32),
            jax.ShapeDtypeStruct((E, T), jnp.float32),
        ],
        scratch_shapes=[
            pltpu.VMEM((H2, T, 2 * HD), jnp.float32),
            pltpu.VMEM((H2, T, 2 * HD), jnp.float32),
            pltpu.VMEM((H2, T, 2 * HD), jnp.float32),
            pltpu.VMEM((H2, T, 2 * HD), jnp.float32),
        ],
    )(x2d, ln1_w.reshape(1, C), ln1_b.reshape(1, C), Wq, bq.reshape(1, C),
      Wk, bk.reshape(1, C), Wv, bv.reshape(1, C), noise_eps,
      Wo, bo.reshape(1, C), ln2_w.reshape(1, C), ln2_b.reshape(1, C),
      Wg, bg.reshape(1, E), Wn, bn_.reshape(1, E))

    router = functools.partial(
        pl.kernel,
        out_type=[
            jax.ShapeDtypeStruct((T,), jnp.int32),
            jax.ShapeDtypeStruct((T,), jnp.int32),
            jax.ShapeDtypeStruct((T,), jnp.float32),
            jax.ShapeDtypeStruct((T,), jnp.float32),
            jax.ShapeDtypeStruct((NB,), jnp.int32),
            jax.ShapeDtypeStruct((NR, C), jnp.float32),
        ],
        mesh=plsc.VectorSubcoreMesh(core_axis_name="c", subcore_axis_name="s",
                                    num_cores=1),
        compiler_params=pltpu.CompilerParams(needs_layout_passes=False),
        scratch_types=[
            pltpu.VMEM((E, TPT), jnp.float32),
            pltpu.VMEM((TPT, C), jnp.float32),
            pltpu.VMEM((2, TPT), jnp.int32),
            pltpu.VMEM((2, TPT), jnp.float32),
            pltpu.VMEM((2, TPT), jnp.int32),
            pltpu.VMEM((2, TPT), jnp.int32),
            pltpu.VMEM((128,), jnp.float32),
            pltpu.VMEM((NT1, 128), jnp.float32),
            pltpu.VMEM((16,), jnp.int32),
            pltpu.VMEM((16,), jnp.int32),
            pltpu.VMEM((NB,), jnp.int32),
            pltpu.VMEM((16,), jnp.float32),
            pltpu.VMEM_SHARED((NT1, 128), jnp.float32),
            pltpu.SemaphoreType.DMA,
            pltpu.SemaphoreType.DMA,
        ],
    )(_router_body)
    pos0, pos1, w0, w1, bex, sorted_rows = router(noisy_t, hs)

    grid_spec = pltpu.PrefetchScalarGridSpec(
        num_scalar_prefetch=1,
        grid=(NB,),
        in_specs=[
            pl.BlockSpec((BR, C), lambda g, b: (g, 0)),
            pl.BlockSpec((1, C, FF), lambda g, b: (b[g], 0, 0)),
            pl.BlockSpec((1, 1, FF), lambda g, b: (b[g], 0, 0)),
            pl.BlockSpec((1, FF, C), lambda g, b: (b[g], 0, 0)),
            pl.BlockSpec((1, 1, C), lambda g, b: (b[g], 0, 0)),
        ],
        out_specs=pl.BlockSpec((BR, C), lambda g, b: (g, 0)),
    )
    outc = pl.pallas_call(
        _ffn_body,
        grid_spec=grid_spec,
        out_shape=jax.ShapeDtypeStruct((NR, C), jnp.float32),
    )(bex, sorted_rows, We1, be1.reshape(E, 1, FF),
      We2, be2.reshape(E, 1, C))

    combine = functools.partial(
        pl.kernel,
        out_type=jax.ShapeDtypeStruct((T, C), jnp.float32),
        mesh=plsc.VectorSubcoreMesh(core_axis_name="c", subcore_axis_name="s"),
        compiler_params=pltpu.CompilerParams(needs_layout_passes=False),
        scratch_types=[
            pltpu.VMEM((TPW,), jnp.int32),
            pltpu.VMEM((TPW,), jnp.int32),
            pltpu.VMEM((TPW,), jnp.float32),
            pltpu.VMEM((TPW,), jnp.float32),
            pltpu.VMEM((HTPW, C), jnp.float32),
            pltpu.VMEM((HTPW, C), jnp.float32),
            pltpu.VMEM((HTPW, C), jnp.float32),
            pltpu.VMEM((HTPW, C), jnp.float32),
            pltpu.SemaphoreType.DMA,
        ],
    )(_combine_body)
    final = combine(x2d, outc, pos0, pos1, w0, w1)

    return (final.reshape(B, T, C), noisy, gate)


# speedup vs baseline: 1.1158x; 1.0227x over previous
import functools
import math

import jax
import jax.numpy as jnp
from jax import lax
from jax.experimental import pallas as pl
from jax.experimental.pallas import tpu as pltpu
from jax.experimental.pallas import tpu_sc as plsc

B, T, C, H, E, K, FF = 1, 2048, 768, 12, 16, 2, 3072
HD = C // H
BT = 1024
BR = 128
NB = T * K // BR + E
NR = NB * BR
NEG = -1e30
NT1 = 16
TPT = T // NT1
NW = 32
TPW = T // NW
HTPW = TPW // 2


H2 = H // 2
NA = T // BT
NBS = H2 * (T // BT)
NC = NA + NBS


def _fused_body(x_ref, ln1w_ref, ln1b_ref, wq_ref, bq_ref, wk_ref, bk_ref,
                wv_ref, bv_ref, eps_ref, wo_ref, bo_ref, ln2w_ref, ln2b_ref,
                wg_ref, bg_ref, wn_ref, bn_ref,
                hs_ref, noisy_ref, gate_ref, noisyT_ref,
                qs, ks, vs, ys):
    s = pl.program_id(0)

    @pl.when(s < NA)
    def _():
        i = s
        x = x_ref[...]
        mu = jnp.mean(x, -1, keepdims=True)
        xc = x - mu
        var = jnp.mean(xc * xc, -1, keepdims=True)
        xn = xc / jnp.sqrt(var + 1e-5) * ln1w_ref[...] + ln1b_ref[...]
        q = jnp.dot(xn, wq_ref[...], preferred_element_type=jnp.float32) + bq_ref[...]
        k = jnp.dot(xn, wk_ref[...], preferred_element_type=jnp.float32) + bk_ref[...]
        v = jnp.dot(xn, wv_ref[...], preferred_element_type=jnp.float32) + bv_ref[...]
        rsl = pl.ds(i * BT, BT)
        for h2 in range(H2):
            csl = slice(h2 * 2 * HD, (h2 + 1) * 2 * HD)
            qs[h2, rsl, :] = q[:, csl]
            ks[h2, rsl, :] = k[:, csl]
            vs[h2, rsl, :] = v[:, csl]

    @pl.when((s >= NA) & (s < NC))
    def _():
        idx = s - NA
        h2d = idx // (T // BT)
        i = idx % (T // BT)
        q2 = qs[h2d, pl.ds(i * BT, BT), :]
        k2 = ks[h2d]
        v2 = vs[h2d]
        rows = jax.lax.broadcasted_iota(jnp.int32, (BT, T), 0) + i * BT
        cols = jax.lax.broadcasted_iota(jnp.int32, (BT, T), 1)
        causal = cols <= rows
        for hh in range(2):
            sl = slice(hh * HD, (hh + 1) * HD)
            sc = jax.lax.dot_general(q2[:, sl], k2[:, sl],
                                     (((1,), (1,)), ((), ())),
                                     preferred_element_type=jnp.float32)
            sc = sc * (1.0 / math.sqrt(HD))
            sc = jnp.where(causal, sc, NEG)
            m = jnp.max(sc, -1, keepdims=True)
            p = jnp.exp(sc - m)
            p = p / jnp.sum(p, -1, keepdims=True)
            ys[h2d, pl.ds(i * BT, BT), sl] = jnp.dot(
                p, v2[:, sl], preferred_element_type=jnp.float32)

    @pl.when(s >= NC)
    def _():
        i = s - NC
        y = jnp.concatenate(
            [ys[h2, pl.ds(i * BT, BT), :] for h2 in range(H2)], axis=1)
        h = x_ref[...] + (jnp.dot(y, wo_ref[...],
                                  preferred_element_type=jnp.float32)
                          + bo_ref[...])
        mu = jnp.mean(h, -1, keepdims=True)
        hc = h - mu
        var = jnp.mean(hc * hc, -1, keepdims=True)
        hs = hc / jnp.sqrt(var + 1e-5) * ln2w_ref[...] + ln2b_ref[...]
        hs_ref[...] = hs
        g = jnp.dot(hs, wg_ref[...], preferred_element_type=jnp.float32) + bg_ref[...]
        nz = jnp.dot(hs, wn_ref[...], preferred_element_type=jnp.float32) + bn_ref[...]
        sp = jnp.maximum(nz, 0.0) + jnp.log1p(jnp.exp(-jnp.abs(nz)))
        gate_ref[...] = g
        noisy = g + eps_ref[...] * sp
        noisy_ref[...] = noisy
        noisyT_ref[...] = noisy.T


def _ffn_body(bex_ref, hs_ref, we1_ref, be1_ref, we2_ref, be2_ref, out_ref):
    t = jnp.dot(hs_ref[...], we1_ref[0],
                preferred_element_type=jnp.float32) + be1_ref[0]
    t = 0.5 * t * (1.0 + jax.lax.erf(t * (1.0 / math.sqrt(2.0))))
    out_ref[...] = jnp.dot(t, we2_ref[0],
                           preferred_element_type=jnp.float32) + be2_ref[0]


def _iota16():
    return jax.lax.broadcasted_iota(jnp.int32, (16,), 0)


def _cumsum16(x, tmp_v, lanes):
    zf = jnp.zeros((16,), jnp.float32)
    for k in (1, 2, 4, 8):
        tmp_v[...] = x
        sh = plsc.load_gather(tmp_v, [jnp.maximum(lanes - k, 0)])
        x = x + jnp.where(lanes >= k, sh, zf)
    return x


def _router_body(nlT_ref, hs_ref,
                 pos0_ref, pos1_ref, w0_ref, w1_ref, bex_ref, sorted_ref,
                 nl_v, hs_v, pose_v, wtsl_v, e_v, rank_v, cnt_v, cnts_all_v,
                 offbase_v, csum_v, bex_v, tmp_v, shared, sem_hs, sem_sc):
    sid = lax.axis_index("s")
    base_tok = sid * TPT
    hs_cp = pltpu.make_async_copy(hs_ref.at[pl.ds(base_tok, TPT)], hs_v, sem_hs)
    hs_cp.start()
    pltpu.sync_copy(nlT_ref.at[:, pl.ds(base_tok, TPT)], nl_v)

    lanes = _iota16()
    cnt = [jnp.float32(0)] * E
    for g in range(TPT // 16):
        sl = pl.ds(g * 16, 16)
        vs = [nl_v[e, sl] for e in range(E)]
        m1 = functools.reduce(jnp.maximum, vs)
        e1v = jnp.full((16,), E, jnp.int32)
        for e in range(E):
            e1v = jnp.minimum(e1v, jnp.where(vs[e] == m1, e, E))
        vs2 = [jnp.where(e1v == e, NEG, vs[e]) for e in range(E)]
        m2 = functools.reduce(jnp.maximum, vs2)
        e2v = jnp.full((16,), E, jnp.int32)
        for e in range(E):
            e2v = jnp.minimum(e2v, jnp.where(vs2[e] == m2, e, E))
        a = jnp.exp(m2 - m1)
        wtsl_v[0, sl] = 1.0 / (1.0 + a)
        wtsl_v[1, sl] = a / (1.0 + a)
        rank0 = jnp.zeros((16,), jnp.float32)
        rank1 = jnp.zeros((16,), jnp.float32)
        for e in range(E):
            m0 = e1v == e
            m1b = e2v == e
            mf = jnp.where(jnp.logical_or(m0, m1b), 1.0, 0.0)
            incl = _cumsum16(mf, tmp_v, lanes)
            r = cnt[e] + (incl - mf)
            rank0 = jnp.where(m0, r, rank0)
            rank1 = jnp.where(m1b, r, rank1)
            cnt[e] = cnt[e] + incl[15]
        e_v[0, sl] = e1v
        e_v[1, sl] = e2v
        rank_v[0, sl] = rank0.astype(jnp.int32)
        rank_v[1, sl] = rank1.astype(jnp.int32)

    cv = jnp.zeros((16,), jnp.float32)
    for e in range(E):
        cv = jnp.where(lanes == e, cnt[e], cv)
    cnt_v[pl.ds(0, 16)] = cv
    zf16 = jnp.zeros((16,), jnp.float32)
    for j in range(1, 8):
        cnt_v[pl.ds(j * 16, 16)] = zf16
    pltpu.sync_copy(cnt_v, shared.at[sid])
    plsc.subcore_barrier()
    pltpu.sync_copy(shared, cnts_all_v)

    sidv = jnp.full((16,), sid, jnp.int32)
    basev = jnp.zeros((16,), jnp.float32)
    totv = jnp.zeros((16,), jnp.float32)
    zf = jnp.zeros((16,), jnp.float32)
    for w2 in range(NT1):
        row = cnts_all_v[w2, pl.ds(0, 16)]
        basev = basev + jnp.where(jnp.full((16,), w2, jnp.int32) < sidv, row, zf)
        totv = totv + row
    pc = ((totv.astype(jnp.int32) + (BR - 1)) >> 7) << 7
    pcf = pc.astype(jnp.float32)
    csum = _cumsum16(pcf, tmp_v, lanes)
    off = csum - pcf
    offbase_v[...] = (off + basev).astype(jnp.int32)
    csum_v[...] = csum.astype(jnp.int32)

    for g in range(TPT // 16):
        sl = pl.ds(g * 16, 16)
        pose_v[0, sl] = plsc.load_gather(offbase_v, [e_v[0, sl]]) + rank_v[0, sl]
        pose_v[1, sl] = plsc.load_gather(offbase_v, [e_v[1, sl]]) + rank_v[1, sl]
    pltpu.sync_copy(pose_v.at[0], pos0_ref.at[pl.ds(base_tok, TPT)])
    pltpu.sync_copy(pose_v.at[1], pos1_ref.at[pl.ds(base_tok, TPT)])
    pltpu.sync_copy(wtsl_v.at[0], w0_ref.at[pl.ds(base_tok, TPT)])
    pltpu.sync_copy(wtsl_v.at[1], w1_ref.at[pl.ds(base_tok, TPT)])

    @pl.when(sid == 0)
    def _():
        csumv = csum_v[...]
        for j in range(NB // 16):
            bvec = lanes + j * 16
            acc = jnp.zeros((16,), jnp.int32)
            for e in range(E):
                end_e = csumv[e] >> 7
                acc = acc + jnp.where(bvec >= end_e, 1, 0).astype(jnp.int32)
            bex_v[pl.ds(j * 16, 16)] = jnp.minimum(acc, E - 1)
        pltpu.sync_copy(bex_v, bex_ref)

    hs_cp.wait()
    c0 = pltpu.make_async_copy(hs_v, sorted_ref.at[pose_v.at[0]], sem_sc)
    c1 = pltpu.make_async_copy(hs_v, sorted_ref.at[pose_v.at[1]], sem_sc)
    c0.start()
    c1.start()
    c0.wait()
    c1.wait()


def _combine_body(x_ref, outc_ref, pos0_ref, pos1_ref, w0_ref, w1_ref,
                  final_ref, pos0_v, pos1_v, w0_v, w1_v, a_v, b_v, x_v, f_v,
                  sem):
    wid = lax.axis_index("s") * 2 + lax.axis_index("c")
    base_tok = wid * TPW
    pltpu.sync_copy(pos0_ref.at[pl.ds(base_tok, TPW)], pos0_v)
    pltpu.sync_copy(pos1_ref.at[pl.ds(base_tok, TPW)], pos1_v)
    pltpu.sync_copy(w0_ref.at[pl.ds(base_tok, TPW)], w0_v)
    pltpu.sync_copy(w1_ref.at[pl.ds(base_tok, TPW)], w1_v)
    for jh in range(2):
        tok0 = base_tok + jh * HTPW
        ca = pltpu.make_async_copy(
            outc_ref.at[pos0_v.at[pl.ds(jh * HTPW, HTPW)]], a_v, sem)
        cb = pltpu.make_async_copy(
            outc_ref.at[pos1_v.at[pl.ds(jh * HTPW, HTPW)]], b_v, sem)
        ca.start()
        cb.start()
        pltpu.sync_copy(x_ref.at[pl.ds(tok0, HTPW)], x_v)
        ca.wait()
        cb.wait()

        def body(t, _):
            ti = jnp.full((16,), jh * HTPW + t, jnp.int32)
            w0 = plsc.load_gather(w0_v, [ti])
            w1 = plsc.load_gather(w1_v, [ti])
            for c in range(C // 16):
                sl = pl.ds(c * 16, 16)
                f_v[t, sl] = x_v[t, sl] + w0 * a_v[t, sl] + w1 * b_v[t, sl]
            return 0

        lax.fori_loop(0, HTPW, body, 0)
        pltpu.sync_copy(f_v, final_ref.at[pl.ds(tok0, HTPW)])


def kernel(hidden_states, ln1_w, ln1_b, ln2_w, ln2_b, Wq, bq, Wk, bk, Wv, bv,
           Wo, bo, Wg, bg, Wn, bn_, We1, be1, We2, be2, noise_eps):
    x2d = hidden_states.reshape(T, C)
    full = lambda r, c: pl.BlockSpec((r, c), lambda *_: (0, 0))

    def _xrow(s):
        return (jnp.where(s < NA, s,
                          jnp.where(s >= NC, s - NC, NA - 1)), 0)

    def _crow(s):
        return (jnp.where(s >= NC, s - NC, 0), 0)

    def _ccol(s):
        return (0, jnp.where(s >= NC, s - NC, 0))

    hs, noisy, gate, noisy_t = pl.pallas_call(
        _fused_body,
        grid=(NC + NA,),
        in_specs=[
            pl.BlockSpec((BT, C), _xrow),
            full(1, C), full(1, C),
            full(C, C), full(1, C), full(C, C), full(1, C),
            full(C, C), full(1, C),
            pl.BlockSpec((BT, E), _crow),
            full(C, C), full(1, C), full(1, C), full(1, C),
            full(C, E), full(1, E), full(C, E), full(1, E),
        ],
        out_specs=[
            pl.BlockSpec((BT, C), _crow),
            pl.BlockSpec((BT, E), _crow),
            pl.BlockSpec((BT, E), _crow),
            pl.BlockSpec((E, BT), _ccol),
        ],
        out_shape=[
            jax.ShapeDtypeStruct((T, C), jnp.float32),
            jax.ShapeDtypeStruct((T, E), jnp.float32),
            jax.ShapeDtypeStruct((T, E), jnp.float32),
            jax.ShapeDtypeStruct((E, T), jnp.float32),
        ],
        scratch_shapes=[
            pltpu.VMEM((H2, T, 2 * HD), jnp.float32),
            pltpu.VMEM((H2, T, 2 * HD), jnp.float32),
            pltpu.VMEM((H2, T, 2 * HD), jnp.float32),
            pltpu.VMEM((H2, T, 2 * HD), jnp.float32),
        ],
    )(x2d, ln1_w.reshape(1, C), ln1_b.reshape(1, C), Wq, bq.reshape(1, C),
      Wk, bk.reshape(1, C), Wv, bv.reshape(1, C), noise_eps,
      Wo, bo.reshape(1, C), ln2_w.reshape(1, C), ln2_b.reshape(1, C),
      Wg, bg.reshape(1, E), Wn, bn_.reshape(1, E))

    router = functools.partial(
        pl.kernel,
        out_type=[
            jax.ShapeDtypeStruct((T,), jnp.int32),
            jax.ShapeDtypeStruct((T,), jnp.int32),
            jax.ShapeDtypeStruct((T,), jnp.float32),
            jax.ShapeDtypeStruct((T,), jnp.float32),
            jax.ShapeDtypeStruct((NB,), jnp.int32),
            jax.ShapeDtypeStruct((NR, C), jnp.float32),
        ],
        mesh=plsc.VectorSubcoreMesh(core_axis_name="c", subcore_axis_name="s",
                                    num_cores=1),
        compiler_params=pltpu.CompilerParams(needs_layout_passes=False),
        scratch_types=[
            pltpu.VMEM((E, TPT), jnp.float32),
            pltpu.VMEM((TPT, C), jnp.float32),
            pltpu.VMEM((2, TPT), jnp.int32),
            pltpu.VMEM((2, TPT), jnp.float32),
            pltpu.VMEM((2, TPT), jnp.int32),
            pltpu.VMEM((2, TPT), jnp.int32),
            pltpu.VMEM((128,), jnp.float32),
            pltpu.VMEM((NT1, 128), jnp.float32),
            pltpu.VMEM((16,), jnp.int32),
            pltpu.VMEM((16,), jnp.int32),
            pltpu.VMEM((NB,), jnp.int32),
            pltpu.VMEM((16,), jnp.float32),
            pltpu.VMEM_SHARED((NT1, 128), jnp.float32),
            pltpu.SemaphoreType.DMA,
            pltpu.SemaphoreType.DMA,
        ],
    )(_router_body)
    pos0, pos1, w0, w1, bex, sorted_rows = router(noisy_t, hs)

    grid_spec = pltpu.PrefetchScalarGridSpec(
        num_scalar_prefetch=1,
        grid=(NB,),
        in_specs=[
            pl.BlockSpec((BR, C), lambda g, b: (g, 0)),
            pl.BlockSpec((1, C, FF), lambda g, b: (b[g], 0, 0)),
            pl.BlockSpec((1, 1, FF), lambda g, b: (b[g], 0, 0)),
            pl.BlockSpec((1, FF, C), lambda g, b: (b[g], 0, 0)),
            pl.BlockSpec((1, 1, C), lambda g, b: (b[g], 0, 0)),
        ],
        out_specs=pl.BlockSpec((BR, C), lambda g, b: (g, 0)),
    )
    outc = pl.pallas_call(
        _ffn_body,
        grid_spec=grid_spec,
        out_shape=jax.ShapeDtypeStruct((NR, C), jnp.float32),
    )(bex, sorted_rows, We1, be1.reshape(E, 1, FF),
      We2, be2.reshape(E, 1, C))

    combine = functools.partial(
        pl.kernel,
        out_type=jax.ShapeDtypeStruct((T, C), jnp.float32),
        mesh=plsc.VectorSubcoreMesh(core_axis_name="c", subcore_axis_name="s"),
        compiler_params=pltpu.CompilerParams(needs_layout_passes=False),
        scratch_types=[
            pltpu.VMEM((TPW,), jnp.int32),
            pltpu.VMEM((TPW,), jnp.int32),
            pltpu.VMEM((TPW,), jnp.float32),
            pltpu.VMEM((TPW,), jnp.float32),
            pltpu.VMEM((HTPW, C), jnp.float32),
            pltpu.VMEM((HTPW, C), jnp.float32),
            pltpu.VMEM((HTPW, C), jnp.float32),
            pltpu.VMEM((HTPW, C), jnp.float32),
            pltpu.SemaphoreType.DMA,
        ],
    )(_combine_body)
    final = combine(x2d, outc, pos0, pos1, w0, w1)

    return (final.reshape(B, T, C), noisy, gate)


# speedup vs baseline: 1.1466x; 1.0276x over previous
import functools
import math

import jax
import jax.numpy as jnp
from jax import lax
from jax.experimental import pallas as pl
from jax.experimental.pallas import tpu as pltpu
from jax.experimental.pallas import tpu_sc as plsc

B, T, C, H, E, K, FF = 1, 2048, 768, 12, 16, 2, 3072
HD = C // H
BT = 1024
BR = 256
BRL = 8
NB = T * K // BR + E
NR = NB * BR
NEG = -1e30
NT1 = 16
TPT = T // NT1
NW = 32
TPW = T // NW
HTPW = TPW // 2


H2 = H // 2
NA = T // BT
NBS = H2 * (T // BT)
NC = NA + NBS


def _fused_body(x_ref, ln1w_ref, ln1b_ref, wq_ref, bq_ref, wk_ref, bk_ref,
                wv_ref, bv_ref, eps_ref, wo_ref, bo_ref, ln2w_ref, ln2b_ref,
                wg_ref, bg_ref, wn_ref, bn_ref,
                hs_ref, noisy_ref, gate_ref, noisyT_ref,
                qs, ks, vs, ys):
    s = pl.program_id(0)

    @pl.when(s < NA)
    def _():
        i = s
        x = x_ref[...]
        mu = jnp.mean(x, -1, keepdims=True)
        xc = x - mu
        var = jnp.mean(xc * xc, -1, keepdims=True)
        xn = xc / jnp.sqrt(var + 1e-5) * ln1w_ref[...] + ln1b_ref[...]
        q = jnp.dot(xn, wq_ref[...], preferred_element_type=jnp.float32) + bq_ref[...]
        k = jnp.dot(xn, wk_ref[...], preferred_element_type=jnp.float32) + bk_ref[...]
        v = jnp.dot(xn, wv_ref[...], preferred_element_type=jnp.float32) + bv_ref[...]
        rsl = pl.ds(i * BT, BT)
        for h2 in range(H2):
            csl = slice(h2 * 2 * HD, (h2 + 1) * 2 * HD)
            qs[h2, rsl, :] = q[:, csl]
            ks[h2, rsl, :] = k[:, csl]
            vs[h2, rsl, :] = v[:, csl]

    @pl.when((s >= NA) & (s < NC))
    def _():
        idx = s - NA
        h2d = idx // (T // BT)
        i = idx % (T // BT)
        q2 = qs[h2d, pl.ds(i * BT, BT), :]
        k2 = ks[h2d]
        v2 = vs[h2d]
        rows = jax.lax.broadcasted_iota(jnp.int32, (BT, T), 0) + i * BT
        cols = jax.lax.broadcasted_iota(jnp.int32, (BT, T), 1)
        causal = cols <= rows
        for hh in range(2):
            sl = slice(hh * HD, (hh + 1) * HD)
            sc = jax.lax.dot_general(q2[:, sl], k2[:, sl],
                                     (((1,), (1,)), ((), ())),
                                     preferred_element_type=jnp.float32)
            sc = sc * (1.0 / math.sqrt(HD))
            sc = jnp.where(causal, sc, NEG)
            m = jnp.max(sc, -1, keepdims=True)
            p = jnp.exp(sc - m)
            p = p / jnp.sum(p, -1, keepdims=True)
            ys[h2d, pl.ds(i * BT, BT), sl] = jnp.dot(
                p, v2[:, sl], preferred_element_type=jnp.float32)

    @pl.when(s >= NC)
    def _():
        i = s - NC
        y = jnp.concatenate(
            [ys[h2, pl.ds(i * BT, BT), :] for h2 in range(H2)], axis=1)
        h = x_ref[...] + (jnp.dot(y, wo_ref[...],
                                  preferred_element_type=jnp.float32)
                          + bo_ref[...])
        mu = jnp.mean(h, -1, keepdims=True)
        hc = h - mu
        var = jnp.mean(hc * hc, -1, keepdims=True)
        hs = hc / jnp.sqrt(var + 1e-5) * ln2w_ref[...] + ln2b_ref[...]
        hs_ref[...] = hs
        g = jnp.dot(hs, wg_ref[...], preferred_element_type=jnp.float32) + bg_ref[...]
        nz = jnp.dot(hs, wn_ref[...], preferred_element_type=jnp.float32) + bn_ref[...]
        sp = jnp.maximum(nz, 0.0) + jnp.log1p(jnp.exp(-jnp.abs(nz)))
        gate_ref[...] = g
        noisy = g + eps_ref[...] * sp
        noisy_ref[...] = noisy
        noisyT_ref[...] = noisy.T


def _ffn_body(bex_ref, hs_ref, we1_ref, be1_ref, we2_ref, be2_ref, out_ref):
    t = jnp.dot(hs_ref[...], we1_ref[0],
                preferred_element_type=jnp.float32) + be1_ref[0]
    t = 0.5 * t * (1.0 + jax.lax.erf(t * (1.0 / math.sqrt(2.0))))
    out_ref[...] = jnp.dot(t, we2_ref[0],
                           preferred_element_type=jnp.float32) + be2_ref[0]


def _iota16():
    return jax.lax.broadcasted_iota(jnp.int32, (16,), 0)


def _cumsum16(x, tmp_v, lanes):
    zf = jnp.zeros((16,), jnp.float32)
    for k in (1, 2, 4, 8):
        tmp_v[...] = x
        sh = plsc.load_gather(tmp_v, [jnp.maximum(lanes - k, 0)])
        x = x + jnp.where(lanes >= k, sh, zf)
    return x


def _router_body(nlT_ref, hs_ref,
                 pos0_ref, pos1_ref, w0_ref, w1_ref, bex_ref, sorted_ref,
                 nl_v, hs_v, pose_v, wtsl_v, e_v, rank_v, cnt_v, cnts_all_v,
                 offbase_v, csum_v, bex_v, tmp_v, shared, sem_hs, sem_sc):
    sid = lax.axis_index("s")
    base_tok = sid * TPT
    hs_cp = pltpu.make_async_copy(hs_ref.at[pl.ds(base_tok, TPT)], hs_v, sem_hs)
    hs_cp.start()
    pltpu.sync_copy(nlT_ref.at[:, pl.ds(base_tok, TPT)], nl_v)

    lanes = _iota16()
    cnt = [jnp.float32(0)] * E
    for g in range(TPT // 16):
        sl = pl.ds(g * 16, 16)
        vs = [nl_v[e, sl] for e in range(E)]
        m1 = functools.reduce(jnp.maximum, vs)
        e1v = jnp.full((16,), E, jnp.int32)
        for e in range(E):
            e1v = jnp.minimum(e1v, jnp.where(vs[e] == m1, e, E))
        vs2 = [jnp.where(e1v == e, NEG, vs[e]) for e in range(E)]
        m2 = functools.reduce(jnp.maximum, vs2)
        e2v = jnp.full((16,), E, jnp.int32)
        for e in range(E):
            e2v = jnp.minimum(e2v, jnp.where(vs2[e] == m2, e, E))
        a = jnp.exp(m2 - m1)
        wtsl_v[0, sl] = 1.0 / (1.0 + a)
        wtsl_v[1, sl] = a / (1.0 + a)
        rank0 = jnp.zeros((16,), jnp.float32)
        rank1 = jnp.zeros((16,), jnp.float32)
        for e in range(E):
            m0 = e1v == e
            m1b = e2v == e
            mf = jnp.where(jnp.logical_or(m0, m1b), 1.0, 0.0)
            incl = _cumsum16(mf, tmp_v, lanes)
            r = cnt[e] + (incl - mf)
            rank0 = jnp.where(m0, r, rank0)
            rank1 = jnp.where(m1b, r, rank1)
            cnt[e] = cnt[e] + incl[15]
        e_v[0, sl] = e1v
        e_v[1, sl] = e2v
        rank_v[0, sl] = rank0.astype(jnp.int32)
        rank_v[1, sl] = rank1.astype(jnp.int32)

    cv = jnp.zeros((16,), jnp.float32)
    for e in range(E):
        cv = jnp.where(lanes == e, cnt[e], cv)
    cnt_v[pl.ds(0, 16)] = cv
    zf16 = jnp.zeros((16,), jnp.float32)
    for j in range(1, 8):
        cnt_v[pl.ds(j * 16, 16)] = zf16
    pltpu.sync_copy(cnt_v, shared.at[sid])
    plsc.subcore_barrier()
    pltpu.sync_copy(shared, cnts_all_v)

    sidv = jnp.full((16,), sid, jnp.int32)
    basev = jnp.zeros((16,), jnp.float32)
    totv = jnp.zeros((16,), jnp.float32)
    zf = jnp.zeros((16,), jnp.float32)
    for w2 in range(NT1):
        row = cnts_all_v[w2, pl.ds(0, 16)]
        basev = basev + jnp.where(jnp.full((16,), w2, jnp.int32) < sidv, row, zf)
        totv = totv + row
    pc = ((totv.astype(jnp.int32) + (BR - 1)) >> BRL) << BRL
    pcf = pc.astype(jnp.float32)
    csum = _cumsum16(pcf, tmp_v, lanes)
    off = csum - pcf
    offbase_v[...] = (off + basev).astype(jnp.int32)
    csum_v[...] = csum.astype(jnp.int32)

    for g in range(TPT // 16):
        sl = pl.ds(g * 16, 16)
        pose_v[0, sl] = plsc.load_gather(offbase_v, [e_v[0, sl]]) + rank_v[0, sl]
        pose_v[1, sl] = plsc.load_gather(offbase_v, [e_v[1, sl]]) + rank_v[1, sl]
    pltpu.sync_copy(pose_v.at[0], pos0_ref.at[pl.ds(base_tok, TPT)])
    pltpu.sync_copy(pose_v.at[1], pos1_ref.at[pl.ds(base_tok, TPT)])
    pltpu.sync_copy(wtsl_v.at[0], w0_ref.at[pl.ds(base_tok, TPT)])
    pltpu.sync_copy(wtsl_v.at[1], w1_ref.at[pl.ds(base_tok, TPT)])

    @pl.when(sid == 0)
    def _():
        csumv = csum_v[...]
        for j in range(NB // 16):
            bvec = lanes + j * 16
            acc = jnp.zeros((16,), jnp.int32)
            for e in range(E):
                end_e = csumv[e] >> BRL
                acc = acc + jnp.where(bvec >= end_e, 1, 0).astype(jnp.int32)
            bex_v[pl.ds(j * 16, 16)] = jnp.minimum(acc, E - 1)
        pltpu.sync_copy(bex_v, bex_ref)

    hs_cp.wait()
    c0 = pltpu.make_async_copy(hs_v, sorted_ref.at[pose_v.at[0]], sem_sc)
    c1 = pltpu.make_async_copy(hs_v, sorted_ref.at[pose_v.at[1]], sem_sc)
    c0.start()
    c1.start()
    c0.wait()
    c1.wait()


def _combine_body(x_ref, outc_ref, pos0_ref, pos1_ref, w0_ref, w1_ref,
                  final_ref, pos0_v, pos1_v, w0_v, w1_v, a_v, b_v, x_v, f_v,
                  sem):
    wid = lax.axis_index("s") * 2 + lax.axis_index("c")
    base_tok = wid * TPW
    pltpu.sync_copy(pos0_ref.at[pl.ds(base_tok, TPW)], pos0_v)
    pltpu.sync_copy(pos1_ref.at[pl.ds(base_tok, TPW)], pos1_v)
    pltpu.sync_copy(w0_ref.at[pl.ds(base_tok, TPW)], w0_v)
    pltpu.sync_copy(w1_ref.at[pl.ds(base_tok, TPW)], w1_v)
    for jh in range(2):
        tok0 = base_tok + jh * HTPW
        ca = pltpu.make_async_copy(
            outc_ref.at[pos0_v.at[pl.ds(jh * HTPW, HTPW)]], a_v, sem)
        cb = pltpu.make_async_copy(
            outc_ref.at[pos1_v.at[pl.ds(jh * HTPW, HTPW)]], b_v, sem)
        ca.start()
        cb.start()
        pltpu.sync_copy(x_ref.at[pl.ds(tok0, HTPW)], x_v)
        ca.wait()
        cb.wait()

        def body(t, _):
            ti = jnp.full((16,), jh * HTPW + t, jnp.int32)
            w0 = plsc.load_gather(w0_v, [ti])
            w1 = plsc.load_gather(w1_v, [ti])
            for c in range(C // 16):
                sl = pl.ds(c * 16, 16)
                f_v[t, sl] = x_v[t, sl] + w0 * a_v[t, sl] + w1 * b_v[t, sl]
            return 0

        lax.fori_loop(0, HTPW, body, 0)
        pltpu.sync_copy(f_v, final_ref.at[pl.ds(tok0, HTPW)])


def kernel(hidden_states, ln1_w, ln1_b, ln2_w, ln2_b, Wq, bq, Wk, bk, Wv, bv,
           Wo, bo, Wg, bg, Wn, bn_, We1, be1, We2, be2, noise_eps):
    x2d = hidden_states.reshape(T, C)
    full = lambda r, c: pl.BlockSpec((r, c), lambda *_: (0, 0))

    def _xrow(s):
        return (jnp.where(s < NA, s,
                          jnp.where(s >= NC, s - NC, NA - 1)), 0)

    def _crow(s):
        return (jnp.where(s >= NC, s - NC, 0), 0)

    def _ccol(s):
        return (0, jnp.where(s >= NC, s - NC, 0))

    hs, noisy, gate, noisy_t = pl.pallas_call(
        _fused_body,
        grid=(NC + NA,),
        in_specs=[
            pl.BlockSpec((BT, C), _xrow),
            full(1, C), full(1, C),
            full(C, C), full(1, C), full(C, C), full(1, C),
            full(C, C), full(1, C),
            pl.BlockSpec((BT, E), _crow),
            full(C, C), full(1, C), full(1, C), full(1, C),
            full(C, E), full(1, E), full(C, E), full(1, E),
        ],
        out_specs=[
            pl.BlockSpec((BT, C), _crow),
            pl.BlockSpec((BT, E), _crow),
            pl.BlockSpec((BT, E), _crow),
            pl.BlockSpec((E, BT), _ccol),
        ],
        out_shape=[
            jax.ShapeDtypeStruct((T, C), jnp.float32),
            jax.ShapeDtypeStruct((T, E), jnp.float32),
            jax.ShapeDtypeStruct((T, E), jnp.float32),
            jax.ShapeDtypeStruct((E, T), jnp.float32),
        ],
        scratch_shapes=[
            pltpu.VMEM((H2, T, 2 * HD), jnp.float32),
            pltpu.VMEM((H2, T, 2 * HD), jnp.float32),
            pltpu.VMEM((H2, T, 2 * HD), jnp.float32),
            pltpu.VMEM((H2, T, 2 * HD), jnp.float32),
        ],
    )(x2d, ln1_w.reshape(1, C), ln1_b.reshape(1, C), Wq, bq.reshape(1, C),
      Wk, bk.reshape(1, C), Wv, bv.reshape(1, C), noise_eps,
      Wo, bo.reshape(1, C), ln2_w.reshape(1, C), ln2_b.reshape(1, C),
      Wg, bg.reshape(1, E), Wn, bn_.reshape(1, E))

    router = functools.partial(
        pl.kernel,
        out_type=[
            jax.ShapeDtypeStruct((T,), jnp.int32),
            jax.ShapeDtypeStruct((T,), jnp.int32),
            jax.ShapeDtypeStruct((T,), jnp.float32),
            jax.ShapeDtypeStruct((T,), jnp.float32),
            jax.ShapeDtypeStruct((NB,), jnp.int32),
            jax.ShapeDtypeStruct((NR, C), jnp.float32),
        ],
        mesh=plsc.VectorSubcoreMesh(core_axis_name="c", subcore_axis_name="s",
                                    num_cores=1),
        compiler_params=pltpu.CompilerParams(needs_layout_passes=False),
        scratch_types=[
            pltpu.VMEM((E, TPT), jnp.float32),
            pltpu.VMEM((TPT, C), jnp.float32),
            pltpu.VMEM((2, TPT), jnp.int32),
            pltpu.VMEM((2, TPT), jnp.float32),
            pltpu.VMEM((2, TPT), jnp.int32),
            pltpu.VMEM((2, TPT), jnp.int32),
            pltpu.VMEM((128,), jnp.float32),
            pltpu.VMEM((NT1, 128), jnp.float32),
            pltpu.VMEM((16,), jnp.int32),
            pltpu.VMEM((16,), jnp.int32),
            pltpu.VMEM((NB,), jnp.int32),
            pltpu.VMEM((16,), jnp.float32),
            pltpu.VMEM_SHARED((NT1, 128), jnp.float32),
            pltpu.SemaphoreType.DMA,
            pltpu.SemaphoreType.DMA,
        ],
    )(_router_body)
    pos0, pos1, w0, w1, bex, sorted_rows = router(noisy_t, hs)

    grid_spec = pltpu.PrefetchScalarGridSpec(
        num_scalar_prefetch=1,
        grid=(NB,),
        in_specs=[
            pl.BlockSpec((BR, C), lambda g, b: (g, 0)),
            pl.BlockSpec((1, C, FF), lambda g, b: (b[g], 0, 0)),
            pl.BlockSpec((1, 1, FF), lambda g, b: (b[g], 0, 0)),
            pl.BlockSpec((1, FF, C), lambda g, b: (b[g], 0, 0)),
            pl.BlockSpec((1, 1, C), lambda g, b: (b[g], 0, 0)),
        ],
        out_specs=pl.BlockSpec((BR, C), lambda g, b: (g, 0)),
    )
    outc = pl.pallas_call(
        _ffn_body,
        grid_spec=grid_spec,
        out_shape=jax.ShapeDtypeStruct((NR, C), jnp.float32),
    )(bex, sorted_rows, We1, be1.reshape(E, 1, FF),
      We2, be2.reshape(E, 1, C))

    combine = functools.partial(
        pl.kernel,
        out_type=jax.ShapeDtypeStruct((T, C), jnp.float32),
        mesh=plsc.VectorSubcoreMesh(core_axis_name="c", subcore_axis_name="s"),
        compiler_params=pltpu.CompilerParams(needs_layout_passes=False),
        scratch_types=[
            pltpu.VMEM((TPW,), jnp.int32),
            pltpu.VMEM((TPW,), jnp.int32),
            pltpu.VMEM((TPW,), jnp.float32),
            pltpu.VMEM((TPW,), jnp.float32),
            pltpu.VMEM((HTPW, C), jnp.float32),
            pltpu.VMEM((HTPW, C), jnp.float32),
            pltpu.VMEM((HTPW, C), jnp.float32),
            pltpu.VMEM((HTPW, C), jnp.float32),
            pltpu.SemaphoreType.DMA,
        ],
    )(_combine_body)
    final = combine(x2d, outc, pos0, pos1, w0, w1)

    return (final.reshape(B, T, C), noisy, gate)
